# Initial kernel scaffold; baseline (speedup 1.0000x reference)
import jax
import jax.numpy as jnp
from jax.experimental import pallas as pl


def kernel(x, edge_attr, prev_h, params, edge_index):
    raise NotImplementedError("write your pallas kernel here")


# speedup vs baseline: 1.2316x; 1.2316x over previous
import functools

import jax
import jax.numpy as jnp
from jax import lax
from jax.experimental import pallas as pl
from jax.experimental.pallas import tpu as pltpu
from jax.experimental.pallas import tpu_sc as plsc

N = 50000
E = 800000
D = 64
ED = 16
IN_DIM = 128
L = 4

NC = 2
NS = 16
NW = NC * NS

CHUNK = 128
EPAD = 6272 * CHUNK
PCH = EPAD // 2
NCHUNK = EPAD // CHUNK
ROUNDS_W = NCHUNK // NW

HALF = N // 2
TBL_P = HALF // 2
TBL_PV = 12544
STBL = 12800
ZCH = 32
ZCOPIES = STBL // NS // ZCH
DR_P = 16
DR_PER_TILE = TBL_PV // NS // DR_P

SCHUNK = 64
SNCHUNK = EPAD // SCHUNK
SROUNDS = SNCHUNK // NS

RB_N = 1000
RB_E = 3136
EP = E // 2
NP = N // 2

_EPS = 1e-5


def _bd2(w):
  k, m = w.shape
  out = jnp.zeros((2 * k, 2 * m), w.dtype)
  out = out.at[:k, :m].set(w)
  out = out.at[k:, m:].set(w)
  return out


def _k_input(x_ref, hp_ref, wx_ref, wh_ref, b_ref, o_ref):
  acc = jnp.dot(x_ref[...], wx_ref[...], preferred_element_type=jnp.float32)
  acc = acc + jnp.dot(hp_ref[...], wh_ref[...],
                      preferred_element_type=jnp.float32)
  o_ref[...] = jnp.maximum(acc + b_ref[...], 0.0)


def _input_proj(xp, hprevp, wx_bd, wh_bd, b128):
  grid = NP // RB_N
  return pl.pallas_call(
      _k_input,
      grid=(grid,),
      in_specs=[
          pl.BlockSpec((RB_N, 2 * IN_DIM), lambda i: (i, 0)),
          pl.BlockSpec((RB_N, 128), lambda i: (i, 0)),
          pl.BlockSpec((2 * IN_DIM, 128), lambda i: (0, 0)),
          pl.BlockSpec((128, 128), lambda i: (0, 0)),
          pl.BlockSpec((1, 128), lambda i: (0, 0)),
      ],
      out_specs=pl.BlockSpec((RB_N, 128), lambda i: (i, 0)),
      out_shape=jax.ShapeDtypeStruct((NP, 128), jnp.float32),
  )(xp, hprevp, wx_bd, wh_bd, b128)


def _k_ab_tbl(h_ref, we_ref, wo_ref, o_ref):
  h = h_ref[...]
  ev = jnp.dot(h, we_ref[...], preferred_element_type=jnp.float32)
  od = jnp.dot(h, wo_ref[...], preferred_element_type=jnp.float32)
  rb = h.shape[0]
  o_ref[...] = jnp.stack([ev, od], axis=1).reshape(2 * rb, 128)


def _precompute_tbl(hp, w_e, w_o):
  grid = NP // RB_N
  return pl.pallas_call(
      _k_ab_tbl,
      grid=(grid,),
      in_specs=[
          pl.BlockSpec((RB_N, 128), lambda i: (i, 0)),
          pl.BlockSpec((128, 128), lambda i: (0, 0)),
          pl.BlockSpec((128, 128), lambda i: (0, 0)),
      ],
      out_specs=pl.BlockSpec((2 * RB_N, 128), lambda i: (i, 0)),
      out_shape=jax.ShapeDtypeStruct((N, 128), jnp.float32),
  )(hp, w_e, w_o)


def _k_mm(x_ref, w_ref, o_ref):
  o_ref[...] = jnp.dot(x_ref[...], w_ref[...],
                       preferred_element_type=jnp.float32)


def _edge_c(eap, wc_bd):
  grid = PCH // RB_E
  return pl.pallas_call(
      _k_mm,
      grid=(grid,),
      in_specs=[
          pl.BlockSpec((RB_E, 2 * ED), lambda i: (i, 0)),
          pl.BlockSpec((2 * ED, 128), lambda i: (0, 0)),
      ],
      out_specs=pl.BlockSpec((RB_E, 128), lambda i: (i, 0)),
      out_shape=jax.ShapeDtypeStruct((PCH, 128), jnp.float32),
  )(eap, wc_bd)


def _k_bn_mm_stats(y_ref, ac_ref, w_ref, o_ref, st_ref, acc_ref, *,
                   steps, rb, valid_rows):
  i = pl.program_id(0)

  @pl.when(i == 0)
  def _():
    acc_ref[...] = jnp.zeros_like(acc_ref)

  a = ac_ref[0:1, :]
  c = ac_ref[1:2, :]
  m = jnp.maximum(y_ref[...] * a + c, 0.0)
  z = jnp.dot(m, w_ref[...], preferred_element_type=jnp.float32)
  o_ref[...] = z
  if valid_rows == steps * rb:
    zm = z
  else:
    row = lax.broadcasted_iota(jnp.int32, (rb, 1), 0) + i * rb
    zm = jnp.where(row < valid_rows, z, 0.0)
  ps = jnp.sum(zm, axis=0)
  qs = jnp.sum(zm * zm, axis=0)
  new0 = acc_ref[0, :] + ps
  new1 = acc_ref[1, :] + qs
  acc_ref[0, :] = new0
  acc_ref[1, :] = new1

  @pl.when(i == steps - 1)
  def _():
    s64 = (new0[:D] + new0[D:])[None, :]
    q64 = (new1[:D] + new1[D:])[None, :]
    st_ref[...] = jnp.concatenate(
        [s64, q64, jnp.zeros((30, D), jnp.float32)], axis=0)


def _bn_mm_stats(yp, ac, w_bd, rows, rb, valid_rows=None):
  steps = rows // rb
  if valid_rows is None:
    valid_rows = rows
  return pl.pallas_call(
      functools.partial(_k_bn_mm_stats, steps=steps, rb=rb,
                        valid_rows=valid_rows),
      grid=(steps,),
      in_specs=[
          pl.BlockSpec((rb, 128), lambda i: (i, 0)),
          pl.BlockSpec((8, 128), lambda i: (0, 0)),
          pl.BlockSpec((128, 128), lambda i: (0, 0)),
      ],
      out_specs=[
          pl.BlockSpec((rb, 128), lambda i: (i, 0)),
          pl.BlockSpec((32, D), lambda i: (0, 0)),
      ],
      out_shape=[
          jax.ShapeDtypeStruct((rows, 128), jnp.float32),
          jax.ShapeDtypeStruct((32, D), jnp.float32),
      ],
      scratch_shapes=[pltpu.VMEM((8, 128), jnp.float32)],
  )(yp, ac, w_bd)


def _k_update1(h_ref, g_ref, wa_ref, wb_ref, o_ref, st_ref, acc_ref, *, steps):
  i = pl.program_id(0)

  @pl.when(i == 0)
  def _():
    acc_ref[...] = jnp.zeros_like(acc_ref)

  z = jnp.dot(h_ref[...], wa_ref[...], preferred_element_type=jnp.float32)
  z = z + jnp.dot(g_ref[...], wb_ref[...], preferred_element_type=jnp.float32)
  o_ref[...] = z
  new0 = acc_ref[0, :] + jnp.sum(z, axis=0)
  new1 = acc_ref[1, :] + jnp.sum(z * z, axis=0)
  acc_ref[0, :] = new0
  acc_ref[1, :] = new1

  @pl.when(i == steps - 1)
  def _():
    s64 = (new0[:D] + new0[D:])[None, :]
    q64 = (new1[:D] + new1[D:])[None, :]
    st_ref[...] = jnp.concatenate(
        [s64, q64, jnp.zeros((30, D), jnp.float32)], axis=0)


def _update1(hp, aggrp, wa_bd, wb_bd):
  steps = NP // RB_N
  return pl.pallas_call(
      functools.partial(_k_update1, steps=steps),
      grid=(steps,),
      in_specs=[
          pl.BlockSpec((RB_N, 128), lambda i: (i, 0)),
          pl.BlockSpec((RB_N, 128), lambda i: (i, 0)),
          pl.BlockSpec((128, 128), lambda i: (0, 0)),
          pl.BlockSpec((128, 128), lambda i: (0, 0)),
      ],
      out_specs=[
          pl.BlockSpec((RB_N, 128), lambda i: (i, 0)),
          pl.BlockSpec((32, D), lambda i: (0, 0)),
      ],
      out_shape=[
          jax.ShapeDtypeStruct((NP, 128), jnp.float32),
          jax.ShapeDtypeStruct((32, D), jnp.float32),
      ],
      scratch_shapes=[pltpu.VMEM((8, 128), jnp.float32)],
  )(hp, aggrp, wa_bd, wb_bd)


def _k_finalize(sq_ref, gb_ref, o_ref, *, count):
  t = jnp.sum(sq_ref[...].reshape(NW * 8, 128), axis=0)
  s = t[:D]
  q = t[D:]
  mu = s / count
  var = q / count - mu * mu
  a = gb_ref[0, :] * lax.rsqrt(var + _EPS)
  c = gb_ref[1, :] - a * mu
  a128 = jnp.concatenate([a, a])[None, :]
  c128 = jnp.concatenate([c, c])[None, :]
  o_ref[...] = jnp.concatenate(
      [a128, c128, jnp.zeros((6, 128), jnp.float32)], axis=0)


def _finalize(sq, gb, count):
  return pl.pallas_call(
      functools.partial(_k_finalize, count=float(count)),
      in_specs=[
          pl.BlockSpec((NW, 8, 128), lambda: (0, 0, 0)),
          pl.BlockSpec((8, D), lambda: (0, 0)),
      ],
      out_specs=pl.BlockSpec((8, 128), lambda: (0, 0)),
      out_shape=jax.ShapeDtypeStruct((8, 128), jnp.float32),
  )(sq, gb)


def _k_finalize_sq(sq_ref, gb_ref, o_ref, *, count):
  s = sq_ref[0, :]
  q = sq_ref[1, :]
  mu = s / count
  var = q / count - mu * mu
  a = gb_ref[0, :] * lax.rsqrt(var + _EPS)
  c = gb_ref[1, :] - a * mu
  a128 = jnp.concatenate([a, a])[None, :]
  c128 = jnp.concatenate([c, c])[None, :]
  o_ref[...] = jnp.concatenate(
      [a128, c128, jnp.zeros((6, 128), jnp.float32)], axis=0)


def _finalize_sq(sq, gb, count):
  return pl.pallas_call(
      functools.partial(_k_finalize_sq, count=float(count)),
      in_specs=[
          pl.BlockSpec((32, D), lambda: (0, 0)),
          pl.BlockSpec((8, D), lambda: (0, 0)),
      ],
      out_specs=pl.BlockSpec((8, 128), lambda: (0, 0)),
      out_shape=jax.ShapeDtypeStruct((8, 128), jnp.float32),
  )(sq, gb)


def _k_resid(h_ref, y_ref, ac_ref, o_ref):
  a = ac_ref[0:1, :]
  c = ac_ref[1:2, :]
  o_ref[...] = h_ref[...] + jnp.maximum(y_ref[...] * a + c, 0.0)


def _resid(hp, yp, ac):
  grid = NP // RB_N
  return pl.pallas_call(
      _k_resid,
      grid=(grid,),
      in_specs=[
          pl.BlockSpec((RB_N, 128), lambda i: (i, 0)),
          pl.BlockSpec((RB_N, 128), lambda i: (i, 0)),
          pl.BlockSpec((8, 128), lambda i: (0, 0)),
      ],
      out_specs=pl.BlockSpec((RB_N, 128), lambda i: (i, 0)),
      out_shape=jax.ShapeDtypeStruct((NP, 128), jnp.float32),
  )(hp, yp, ac)


def _k_heads(h_ref, wn_ref, no_ref, hs_ref, acc_ref, *, steps):
  i = pl.program_id(0)

  @pl.when(i == 0)
  def _():
    acc_ref[...] = jnp.zeros_like(acc_ref)

  h = h_ref[...]
  no_ref[...] = jnp.dot(h, wn_ref[...], preferred_element_type=jnp.float32)
  new0 = acc_ref[0, :] + jnp.sum(h, axis=0)
  acc_ref[0, :] = new0

  @pl.when(i == steps - 1)
  def _():
    s64 = (new0[:D] + new0[D:])[None, :]
    hs_ref[...] = jnp.concatenate(
        [s64, jnp.zeros((31, D), jnp.float32)], axis=0)


def _heads(hp, wn_bd, nb128):
  steps = NP // RB_N
  nop, hs = pl.pallas_call(
      functools.partial(_k_heads, steps=steps),
      grid=(steps,),
      in_specs=[
          pl.BlockSpec((RB_N, 128), lambda i: (i, 0)),
          pl.BlockSpec((128, 2 * 5), lambda i: (0, 0)),
      ],
      out_specs=[
          pl.BlockSpec((RB_N, 2 * 5), lambda i: (i, 0)),
          pl.BlockSpec((32, D), lambda i: (0, 0)),
      ],
      out_shape=[
          jax.ShapeDtypeStruct((NP, 2 * 5), jnp.float32),
          jax.ShapeDtypeStruct((32, D), jnp.float32),
      ],
      scratch_shapes=[pltpu.VMEM((8, 128), jnp.float32)],
  )(hp, wn_bd)
  nop = (nop + nb128).reshape(N, 5)
  return nop, hs


def _k_graph(hs_ref, gp_ref, o_ref):
  hmean = jnp.sum(hs_ref[...], axis=0) / float(N)
  g = jnp.sum(hmean * gp_ref[0, :]) + gp_ref[1, 0]
  o_ref[...] = jnp.full((8, 128), g, jnp.float32)


def _graph_head(hs, gp):
  out = pl.pallas_call(
      _k_graph,
      in_specs=[
          pl.BlockSpec((32, D), lambda: (0, 0)),
          pl.BlockSpec((8, D), lambda: (0, 0)),
      ],
      out_specs=pl.BlockSpec((8, 128), lambda: (0, 0)),
      out_shape=jax.ShapeDtypeStruct((8, 128), jnp.float32),
  )(hs, gp)
  return out[0, 0:1]


def _sc_combine_body(ab_hbm, c_hbm, dst_hbm, src_hbm,
                     y_hbm, sq_hbm,
                     idx_d, idx_s, ga, gb, cy, sqb,
                     sem1, sem2):
  wid = lax.axis_index("s") * NC + lax.axis_index("c")

  for r in range(8):
    for g in range(8):
      sqb[r, pl.ds(g * 16, 16)] = jnp.zeros((16,), jnp.float32)

  def round_body(i, carry):
    cid = i * NW + wid
    base = cid * CHUNK
    pbase = cid * (CHUNK // 2)
    pltpu.sync_copy(dst_hbm.at[pl.ds(base, CHUNK)], idx_d)
    pltpu.sync_copy(src_hbm.at[pl.ds(base, CHUNK)], idx_s)
    pltpu.async_copy(ab_hbm.at[idx_d], ga, sem1).wait()
    pltpu.async_copy(ab_hbm.at[idx_s], gb, sem2).wait()
    pltpu.sync_copy(c_hbm.at[pl.ds(pbase, CHUNK // 2)], cy)

    def row_body(r, acc):
      acc = list(acc)
      for par in range(2):
        j = 2 * r + par
        for g in range(4):
          sl = pl.ds(g * 16, 16)
          slb = pl.ds(D + g * 16, 16)
          slc = pl.ds(par * D + g * 16, 16)
          v = ga[j, sl] + gb[j, slb] + cy[r, slc]
          cy[r, slc] = v
          acc[g] = acc[g] + v
          acc[4 + g] = acc[4 + g] + v * v
      return tuple(acc)

    z = jnp.zeros((16,), jnp.float32)
    acc = lax.fori_loop(0, CHUNK // 2, row_body,
                        (z, z, z, z, z, z, z, z))
    for g in range(8):
      sl = pl.ds(g * 16, 16)
      sqb[0, sl] = sqb[0, sl] + acc[g]

    pltpu.sync_copy(cy, y_hbm.at[pl.ds(pbase, CHUNK // 2)])
    return carry

  lax.fori_loop(0, ROUNDS_W, round_body, 0)

  pltpu.sync_copy(sqb, sq_hbm.at[wid])


def _sc_combine(ab_tbl, cp, dst, src):
  mesh = plsc.VectorSubcoreMesh(core_axis_name="c", subcore_axis_name="s",
                                num_cores=NC, num_subcores=NS)
  f = pl.kernel(
      _sc_combine_body,
      out_type=[
          jax.ShapeDtypeStruct((PCH, 128), jnp.float32),
          jax.ShapeDtypeStruct((NW, 8, 128), jnp.float32),
      ],
      mesh=mesh,
      scratch_types=[
          pltpu.VMEM((CHUNK,), jnp.int32),
          pltpu.VMEM((CHUNK,), jnp.int32),
          pltpu.VMEM((CHUNK, 128), jnp.float32),
          pltpu.VMEM((CHUNK, 128), jnp.float32),
          pltpu.VMEM((CHUNK // 2, 128), jnp.float32),
          pltpu.VMEM((8, 128), jnp.float32),
          pltpu.SemaphoreType.DMA,
          pltpu.SemaphoreType.DMA,
      ],
  )
  return f(ab_tbl, cp, dst, src)


def _sc_scatter_body(y_hbm, dst_hbm, ac_hbm, out_hbm,
                     tbl, idxe, idxo, ybuf, rowsl, rowsr, zb, acb, sem1):
  c = lax.axis_index("c")
  s = lax.axis_index("s")
  lo = c * HALF

  def zrow(j, carry):
    for g in range(8):
      sl = pl.ds(g * 16, 16)
      zb[j, sl] = jnp.zeros((16,), jnp.float32)
      rowsl[j, sl] = jnp.zeros((16,), jnp.float32)
      rowsr[j, sl] = jnp.zeros((16,), jnp.float32)
    return carry
  lax.fori_loop(0, ZCH, zrow, 0)

  def zrow2(j, carry):
    for g in range(8):
      sl = pl.ds(g * 16, 16)
      rowsl[ZCH + j, sl] = jnp.zeros((16,), jnp.float32)
      rowsr[ZCH + j, sl] = jnp.zeros((16,), jnp.float32)
    return carry
  lax.fori_loop(0, SCHUNK - ZCH, zrow2, 0)

  def zcopy(k, carry):
    pltpu.sync_copy(zb, tbl.at[pl.ds((s * ZCOPIES + k) * ZCH, ZCH)])
    return carry
  lax.fori_loop(0, ZCOPIES, zcopy, 0)

  pltpu.sync_copy(ac_hbm, acb)

  plsc.subcore_barrier()

  a_regs = [acb[0, pl.ds(g * 16, 16)] for g in range(4)]
  c_regs = [acb[1, pl.ds(g * 16, 16)] for g in range(4)]
  lane = lax.iota(jnp.int32, 16)

  def round_body(i, carry):
    cid = i * NS + s
    base = cid * SCHUNK
    pbase = cid * (SCHUNK // 2)
    pltpu.sync_copy(dst_hbm.at[pl.ds(base, SCHUNK)], idxe)
    pltpu.async_copy(y_hbm.at[pl.ds(pbase, SCHUNK // 2)], ybuf, sem1).wait()

    for jj in range(SCHUNK // 16):
      sl = pl.ds(jj * 16, 16)
      v = idxe[sl]
      valid = (v >= lo) & (v < lo + HALF)
      local = v - lo
      packed = lax.shift_right_logical(local, 1)
      oddb = lax.bitwise_and(local, 1)
      dump = TBL_P + lane + 16 * jj
      dump2 = dump + 64
      evens = packed + oddb * (dump - packed)
      odds = packed + (1 - oddb) * (dump2 - packed)
      idxe[sl] = jnp.where(valid, evens, dump)
      idxo[sl] = jnp.where(valid, odds, dump2)

    def row_body(r, carry2):
      for par in range(2):
        j = 2 * r + par
        for g in range(4):
          v = ybuf[r, pl.ds(par * D + g * 16, 16)]
          mv = jnp.maximum(v * a_regs[g] + c_regs[g], 0.0)
          rowsl[j, pl.ds(g * 16, 16)] = mv
          rowsr[j, pl.ds(D + g * 16, 16)] = mv
      return carry2
    lax.fori_loop(0, SCHUNK // 2, row_body, 0)

    pltpu.sync_copy(rowsl, tbl.at[idxe], add=True)
    pltpu.sync_copy(rowsr, tbl.at[idxo], add=True)
    return carry

  lax.fori_loop(0, SROUNDS, round_body, 0)

  plsc.subcore_barrier()

  def drain_body(k, carry):
    prb = s * (TBL_PV // NS) + k * DR_P
    pltpu.sync_copy(tbl.at[pl.ds(prb, DR_P)],
                    out_hbm.at[pl.ds(c * TBL_PV + prb, DR_P)])
    return carry

  lax.fori_loop(0, DR_PER_TILE, drain_body, 0)


def _sc_scatter(y_edge, dst, ac):
  mesh = plsc.VectorSubcoreMesh(core_axis_name="c", subcore_axis_name="s",
                                num_cores=NC, num_subcores=NS)
  f = pl.kernel(
      _sc_scatter_body,
      out_type=jax.ShapeDtypeStruct((2 * TBL_PV, 128), jnp.float32),
      mesh=mesh,
      scratch_types=[
          pltpu.VMEM_SHARED((STBL, 128), jnp.float32),
          pltpu.VMEM((SCHUNK,), jnp.int32),
          pltpu.VMEM((SCHUNK,), jnp.int32),
          pltpu.VMEM((SCHUNK // 2, 128), jnp.float32),
          pltpu.VMEM((SCHUNK, 128), jnp.float32),
          pltpu.VMEM((SCHUNK, 128), jnp.float32),
          pltpu.VMEM((ZCH, 128), jnp.float32),
          pltpu.VMEM((8, 128), jnp.float32),
          pltpu.SemaphoreType.DMA,
      ],
  )
  return f(y_edge, dst, ac)


def _sc_combine_jnp(ab_tbl, cp, dst, src):
  a_t = ab_tbl[:, :D]
  b_t = ab_tbl[:, D:]
  c = cp.reshape(EPAD, D)
  y1 = a_t[dst] + b_t[src] + c
  sq = (jnp.zeros((NW, 8, 128), jnp.float32)
        .at[0, 0, :D].set(y1.sum(0))
        .at[0, 0, D:].set((y1 * y1).sum(0)))
  return y1.reshape(PCH, 128), sq


def _sc_scatter_jnp(y2p, dst, ac):
  y = y2p.reshape(EPAD, D)
  a = ac[0, :D]
  c = ac[1, :D]
  m = jnp.maximum(y * a + c, 0.0)
  aggr = jax.ops.segment_sum(m, dst, num_segments=N).reshape(NP, 128)
  out = jnp.zeros((2 * TBL_PV, 128), jnp.float32)
  out = out.at[:HALF // 2].set(aggr[:HALF // 2])
  return out.at[TBL_PV:TBL_PV + HALF // 2].set(aggr[HALF // 2:])


def _pad_gb(g, b):
  gb = jnp.zeros((8, D), jnp.float32)
  return gb.at[0, :].set(g).at[1, :].set(b)


def kernel(x, edge_attr, prev_h, params, edge_index):
  pad = jnp.full((EPAD - E,), N, jnp.int32)
  src = jnp.concatenate([edge_index[0], pad])
  dst = jnp.concatenate([edge_index[1], pad])

  xp = x.reshape(NP, 2 * IN_DIM)
  hprevp = prev_h.reshape(NP, 128)
  eap = jnp.zeros((PCH, 2 * ED), jnp.float32).at[:EP].set(
      edge_attr.reshape(EP, 2 * ED))

  b_in_hist = params['b_in'] + params['b_hist']
  b128 = jnp.tile(b_in_hist, 2)[None, :]

  hp = _input_proj(xp, hprevp, _bd2(params['W_in']), _bd2(params['W_hist']),
                   b128)

  for l in range(L):
    w1 = params['msg_W1'][l]
    w1a, w1b, w1c = w1[:D], w1[D:2 * D], w1[2 * D:]

    wab = jnp.concatenate([w1a, w1b], axis=1)
    w_e = jnp.zeros((128, 128), jnp.float32).at[:D, :].set(wab)
    w_o = jnp.zeros((128, 128), jnp.float32).at[D:, :].set(wab)

    ab_tbl = _precompute_tbl(hp, w_e, w_o)
    tblp = jnp.zeros((N + 8, 128), jnp.float32).at[:N].set(ab_tbl)
    cp = _edge_c(eap, _bd2(w1c))

    y1, sq1 = _sc_combine(tblp, cp, dst, src)

    ac1 = _finalize(sq1,
                    _pad_gb(params['msg_g1'][l], params['msg_be1'][l]), E)
    y2p, sq2 = _bn_mm_stats(y1, ac1,
                            _bd2(params['msg_W2'][l]), PCH, RB_E,
                            valid_rows=EP)
    ac2 = _finalize_sq(sq2,
                       _pad_gb(params['msg_g2'][l], params['msg_be2'][l]), E)

    slab = _sc_scatter(y2p, dst, ac2)
    aggrp = jnp.concatenate(
        [slab[:HALF // 2], slab[TBL_PV:TBL_PV + HALF // 2]], axis=0)

    u1 = params['upd_W1'][l]
    y3p, sq3 = _update1(hp, aggrp, _bd2(u1[:D]), _bd2(u1[D:]))
    ac3 = _finalize_sq(sq3,
                       _pad_gb(params['upd_g1'][l], params['upd_be1'][l]), N)
    y4p, sq4 = _bn_mm_stats(y3p, ac3, _bd2(params['upd_W2'][l]), NP, RB_N)
    ac4 = _finalize_sq(sq4,
                       _pad_gb(params['upd_g2'][l], params['upd_be2'][l]), N)
    hp = _resid(hp, y4p, ac4)

  wn_bd = _bd2(params['Wn'])
  nb128 = jnp.tile(params['bn'], 2)[None, :]
  node_out, hs = _heads(hp, wn_bd, nb128)

  gp = jnp.zeros((8, D), jnp.float32)
  gp = gp.at[0, :].set(params['Wg'][:, 0]).at[1, 0].set(params['bg'][0])
  graph_out = _graph_head(hs, gp)

  return (graph_out, node_out, hp.reshape(N, D))


# speedup vs baseline: 1.4555x; 1.1819x over previous
import functools

import jax
import jax.numpy as jnp
from jax import lax
from jax.experimental import pallas as pl
from jax.experimental.pallas import tpu as pltpu
from jax.experimental.pallas import tpu_sc as plsc

N = 50000
E = 800000
D = 64
ED = 16
IN_DIM = 128
L = 4

NC = 2
NS = 16
NW = NC * NS

CHUNK = 128
EPAD = 6272 * CHUNK
PCH = EPAD // 2
NCHUNK = EPAD // CHUNK
ROUNDS_W = NCHUNK // NW

HALF = N // 2
TBL_P = HALF // 2
TBL_PV = 12544
STBL = 12800
ZCH = 32
ZCOPIES = STBL // NS // ZCH
DR_P = 16
DR_PER_TILE = TBL_PV // NS // DR_P

SCHUNK = 64
SNCHUNK = EPAD // SCHUNK
SROUNDS = SNCHUNK // NS

RB_N = 1000
RB_E = 3136
EP = E // 2
NP = N // 2

_EPS = 1e-5


def _bd2(w):
  k, m = w.shape
  out = jnp.zeros((2 * k, 2 * m), w.dtype)
  out = out.at[:k, :m].set(w)
  out = out.at[k:, m:].set(w)
  return out


def _k_input(x_ref, hp_ref, wx_ref, wh_ref, b_ref, o_ref):
  acc = jnp.dot(x_ref[...], wx_ref[...], preferred_element_type=jnp.float32)
  acc = acc + jnp.dot(hp_ref[...], wh_ref[...],
                      preferred_element_type=jnp.float32)
  o_ref[...] = jnp.maximum(acc + b_ref[...], 0.0)


def _input_proj(xp, hprevp, wx_bd, wh_bd, b128):
  grid = NP // RB_N
  return pl.pallas_call(
      _k_input,
      grid=(grid,),
      in_specs=[
          pl.BlockSpec((RB_N, 2 * IN_DIM), lambda i: (i, 0)),
          pl.BlockSpec((RB_N, 128), lambda i: (i, 0)),
          pl.BlockSpec((2 * IN_DIM, 128), lambda i: (0, 0)),
          pl.BlockSpec((128, 128), lambda i: (0, 0)),
          pl.BlockSpec((1, 128), lambda i: (0, 0)),
      ],
      out_specs=pl.BlockSpec((RB_N, 128), lambda i: (i, 0)),
      out_shape=jax.ShapeDtypeStruct((NP, 128), jnp.float32),
  )(xp, hprevp, wx_bd, wh_bd, b128)


def _k_ab_tbl(h_ref, we_ref, wo_ref, o_ref):
  h = h_ref[...]
  ev = jnp.dot(h, we_ref[...], preferred_element_type=jnp.float32)
  od = jnp.dot(h, wo_ref[...], preferred_element_type=jnp.float32)
  rb = h.shape[0]
  o_ref[...] = jnp.stack([ev, od], axis=1).reshape(2 * rb, 128)


def _precompute_tbl(hp, w_e, w_o):
  grid = NP // RB_N
  return pl.pallas_call(
      _k_ab_tbl,
      grid=(grid,),
      in_specs=[
          pl.BlockSpec((RB_N, 128), lambda i: (i, 0)),
          pl.BlockSpec((128, 128), lambda i: (0, 0)),
          pl.BlockSpec((128, 128), lambda i: (0, 0)),
      ],
      out_specs=pl.BlockSpec((2 * RB_N, 128), lambda i: (i, 0)),
      out_shape=jax.ShapeDtypeStruct((N, 128), jnp.float32),
  )(hp, w_e, w_o)


def _k_mm(x_ref, w_ref, o_ref):
  o_ref[...] = jnp.dot(x_ref[...], w_ref[...],
                       preferred_element_type=jnp.float32)


def _edge_c(eap, wc_bd):
  grid = PCH // RB_E
  return pl.pallas_call(
      _k_mm,
      grid=(grid,),
      in_specs=[
          pl.BlockSpec((RB_E, 2 * ED), lambda i: (i, 0)),
          pl.BlockSpec((2 * ED, 128), lambda i: (0, 0)),
      ],
      out_specs=pl.BlockSpec((RB_E, 128), lambda i: (i, 0)),
      out_shape=jax.ShapeDtypeStruct((PCH, 128), jnp.float32),
  )(eap, wc_bd)


def _k_bn_mm_stats(y_ref, ac_ref, w_ref, o_ref, st_ref, acc_ref, *,
                   steps, rb, valid_rows):
  i = pl.program_id(0)

  @pl.when(i == 0)
  def _():
    acc_ref[...] = jnp.zeros_like(acc_ref)

  a = ac_ref[0:1, :]
  c = ac_ref[1:2, :]
  m = jnp.maximum(y_ref[...] * a + c, 0.0)
  z = jnp.dot(m, w_ref[...], preferred_element_type=jnp.float32)
  o_ref[...] = z
  if valid_rows == steps * rb:
    zm = z
  else:
    row = lax.broadcasted_iota(jnp.int32, (rb, 1), 0) + i * rb
    zm = jnp.where(row < valid_rows, z, 0.0)
  ps = jnp.sum(zm, axis=0)
  qs = jnp.sum(zm * zm, axis=0)
  new0 = acc_ref[0, :] + ps
  new1 = acc_ref[1, :] + qs
  acc_ref[0, :] = new0
  acc_ref[1, :] = new1

  @pl.when(i == steps - 1)
  def _():
    s64 = (new0[:D] + new0[D:])[None, :]
    q64 = (new1[:D] + new1[D:])[None, :]
    st_ref[...] = jnp.concatenate(
        [s64, q64, jnp.zeros((30, D), jnp.float32)], axis=0)


def _bn_mm_stats(yp, ac, w_bd, rows, rb, valid_rows=None):
  steps = rows // rb
  if valid_rows is None:
    valid_rows = rows
  return pl.pallas_call(
      functools.partial(_k_bn_mm_stats, steps=steps, rb=rb,
                        valid_rows=valid_rows),
      grid=(steps,),
      in_specs=[
          pl.BlockSpec((rb, 128), lambda i: (i, 0)),
          pl.BlockSpec((8, 128), lambda i: (0, 0)),
          pl.BlockSpec((128, 128), lambda i: (0, 0)),
      ],
      out_specs=[
          pl.BlockSpec((rb, 128), lambda i: (i, 0)),
          pl.BlockSpec((32, D), lambda i: (0, 0)),
      ],
      out_shape=[
          jax.ShapeDtypeStruct((rows, 128), jnp.float32),
          jax.ShapeDtypeStruct((32, D), jnp.float32),
      ],
      scratch_shapes=[pltpu.VMEM((8, 128), jnp.float32)],
  )(yp, ac, w_bd)


def _k_update1(h_ref, g_ref, wa_ref, wb_ref, o_ref, st_ref, acc_ref, *, steps):
  i = pl.program_id(0)

  @pl.when(i == 0)
  def _():
    acc_ref[...] = jnp.zeros_like(acc_ref)

  z = jnp.dot(h_ref[...], wa_ref[...], preferred_element_type=jnp.float32)
  z = z + jnp.dot(g_ref[...], wb_ref[...], preferred_element_type=jnp.float32)
  o_ref[...] = z
  new0 = acc_ref[0, :] + jnp.sum(z, axis=0)
  new1 = acc_ref[1, :] + jnp.sum(z * z, axis=0)
  acc_ref[0, :] = new0
  acc_ref[1, :] = new1

  @pl.when(i == steps - 1)
  def _():
    s64 = (new0[:D] + new0[D:])[None, :]
    q64 = (new1[:D] + new1[D:])[None, :]
    st_ref[...] = jnp.concatenate(
        [s64, q64, jnp.zeros((30, D), jnp.float32)], axis=0)


def _update1(hp, aggrp, wa_bd, wb_bd):
  steps = NP // RB_N
  return pl.pallas_call(
      functools.partial(_k_update1, steps=steps),
      grid=(steps,),
      in_specs=[
          pl.BlockSpec((RB_N, 128), lambda i: (i, 0)),
          pl.BlockSpec((RB_N, 128), lambda i: (i, 0)),
          pl.BlockSpec((128, 128), lambda i: (0, 0)),
          pl.BlockSpec((128, 128), lambda i: (0, 0)),
      ],
      out_specs=[
          pl.BlockSpec((RB_N, 128), lambda i: (i, 0)),
          pl.BlockSpec((32, D), lambda i: (0, 0)),
      ],
      out_shape=[
          jax.ShapeDtypeStruct((NP, 128), jnp.float32),
          jax.ShapeDtypeStruct((32, D), jnp.float32),
      ],
      scratch_shapes=[pltpu.VMEM((8, 128), jnp.float32)],
  )(hp, aggrp, wa_bd, wb_bd)


def _k_finalize(sq_ref, gb_ref, o_ref, *, count):
  t = jnp.sum(sq_ref[...].reshape(NW * 8, 128), axis=0)
  s = t[:D]
  q = t[D:]
  mu = s / count
  var = q / count - mu * mu
  a = gb_ref[0, :] * lax.rsqrt(var + _EPS)
  c = gb_ref[1, :] - a * mu
  a128 = jnp.concatenate([a, a])[None, :]
  c128 = jnp.concatenate([c, c])[None, :]
  o_ref[...] = jnp.concatenate(
      [a128, c128, jnp.zeros((6, 128), jnp.float32)], axis=0)


def _finalize(sq, gb, count):
  return pl.pallas_call(
      functools.partial(_k_finalize, count=float(count)),
      in_specs=[
          pl.BlockSpec((NW, 8, 128), lambda: (0, 0, 0)),
          pl.BlockSpec((8, D), lambda: (0, 0)),
      ],
      out_specs=pl.BlockSpec((8, 128), lambda: (0, 0)),
      out_shape=jax.ShapeDtypeStruct((8, 128), jnp.float32),
  )(sq, gb)


def _k_finalize_sq(sq_ref, gb_ref, o_ref, *, count):
  s = sq_ref[0, :]
  q = sq_ref[1, :]
  mu = s / count
  var = q / count - mu * mu
  a = gb_ref[0, :] * lax.rsqrt(var + _EPS)
  c = gb_ref[1, :] - a * mu
  a128 = jnp.concatenate([a, a])[None, :]
  c128 = jnp.concatenate([c, c])[None, :]
  o_ref[...] = jnp.concatenate(
      [a128, c128, jnp.zeros((6, 128), jnp.float32)], axis=0)


def _finalize_sq(sq, gb, count):
  return pl.pallas_call(
      functools.partial(_k_finalize_sq, count=float(count)),
      in_specs=[
          pl.BlockSpec((32, D), lambda: (0, 0)),
          pl.BlockSpec((8, D), lambda: (0, 0)),
      ],
      out_specs=pl.BlockSpec((8, 128), lambda: (0, 0)),
      out_shape=jax.ShapeDtypeStruct((8, 128), jnp.float32),
  )(sq, gb)


def _k_resid(h_ref, y_ref, ac_ref, o_ref):
  a = ac_ref[0:1, :]
  c = ac_ref[1:2, :]
  o_ref[...] = h_ref[...] + jnp.maximum(y_ref[...] * a + c, 0.0)


def _resid(hp, yp, ac):
  grid = NP // RB_N
  return pl.pallas_call(
      _k_resid,
      grid=(grid,),
      in_specs=[
          pl.BlockSpec((RB_N, 128), lambda i: (i, 0)),
          pl.BlockSpec((RB_N, 128), lambda i: (i, 0)),
          pl.BlockSpec((8, 128), lambda i: (0, 0)),
      ],
      out_specs=pl.BlockSpec((RB_N, 128), lambda i: (i, 0)),
      out_shape=jax.ShapeDtypeStruct((NP, 128), jnp.float32),
  )(hp, yp, ac)


def _k_heads(h_ref, wn_ref, no_ref, hs_ref, acc_ref, *, steps):
  i = pl.program_id(0)

  @pl.when(i == 0)
  def _():
    acc_ref[...] = jnp.zeros_like(acc_ref)

  h = h_ref[...]
  no_ref[...] = jnp.dot(h, wn_ref[...], preferred_element_type=jnp.float32)
  new0 = acc_ref[0, :] + jnp.sum(h, axis=0)
  acc_ref[0, :] = new0

  @pl.when(i == steps - 1)
  def _():
    s64 = (new0[:D] + new0[D:])[None, :]
    hs_ref[...] = jnp.concatenate(
        [s64, jnp.zeros((31, D), jnp.float32)], axis=0)


def _heads(hp, wn_bd, nb128):
  steps = NP // RB_N
  nop, hs = pl.pallas_call(
      functools.partial(_k_heads, steps=steps),
      grid=(steps,),
      in_specs=[
          pl.BlockSpec((RB_N, 128), lambda i: (i, 0)),
          pl.BlockSpec((128, 2 * 5), lambda i: (0, 0)),
      ],
      out_specs=[
          pl.BlockSpec((RB_N, 2 * 5), lambda i: (i, 0)),
          pl.BlockSpec((32, D), lambda i: (0, 0)),
      ],
      out_shape=[
          jax.ShapeDtypeStruct((NP, 2 * 5), jnp.float32),
          jax.ShapeDtypeStruct((32, D), jnp.float32),
      ],
      scratch_shapes=[pltpu.VMEM((8, 128), jnp.float32)],
  )(hp, wn_bd)
  nop = (nop + nb128).reshape(N, 5)
  return nop, hs


def _k_graph(hs_ref, gp_ref, o_ref):
  hmean = jnp.sum(hs_ref[...], axis=0) / float(N)
  g = jnp.sum(hmean * gp_ref[0, :]) + gp_ref[1, 0]
  o_ref[...] = jnp.full((8, 128), g, jnp.float32)


def _graph_head(hs, gp):
  out = pl.pallas_call(
      _k_graph,
      in_specs=[
          pl.BlockSpec((32, D), lambda: (0, 0)),
          pl.BlockSpec((8, D), lambda: (0, 0)),
      ],
      out_specs=pl.BlockSpec((8, 128), lambda: (0, 0)),
      out_shape=jax.ShapeDtypeStruct((8, 128), jnp.float32),
  )(hs, gp)
  return out[0, 0:1]


def _sc_combine_body(ab_hbm, c_hbm, dst_hbm, src_hbm,
                     y_hbm, sq_hbm,
                     idx_d0, idx_d1, idx_s0, idx_s1,
                     ga0, ga1, gb0, gb1, cb0, cb1, yb0, yb1, sqb,
                     sid0, sid1, sis0, sis1, sga0, sga1, sgb0, sgb1,
                     scc0, scc1, sw0, sw1):
  wid = lax.axis_index("s") * NC + lax.axis_index("c")
  idx_d = [idx_d0, idx_d1]
  idx_s = [idx_s0, idx_s1]
  ga = [ga0, ga1]
  gb = [gb0, gb1]
  cb = [cb0, cb1]
  yb = [yb0, yb1]
  sid = [sid0, sid1]
  sis = [sis0, sis1]
  sga = [sga0, sga1]
  sgb = [sgb0, sgb1]
  scc = [scc0, scc1]
  sw = [sw0, sw1]

  for r in range(8):
    for g in range(8):
      sqb[r, pl.ds(g * 16, 16)] = jnp.zeros((16,), jnp.float32)

  def issue_inputs(r, p):
    cid = r * NW + wid
    base = cid * CHUNK
    pbase = cid * (CHUNK // 2)
    pltpu.async_copy(dst_hbm.at[pl.ds(base, CHUNK)], idx_d[p], sid[p])
    pltpu.async_copy(src_hbm.at[pl.ds(base, CHUNK)], idx_s[p], sis[p])
    pltpu.async_copy(c_hbm.at[pl.ds(pbase, CHUNK // 2)], cb[p], scc[p])

  def wait_idx(p):
    pltpu.make_async_copy(dst_hbm.at[pl.ds(0, CHUNK)], idx_d[p],
                          sid[p]).wait()
    pltpu.make_async_copy(src_hbm.at[pl.ds(0, CHUNK)], idx_s[p],
                          sis[p]).wait()

  def issue_gathers(p):
    pltpu.async_copy(ab_hbm.at[idx_d[p]], ga[p], sga[p])
    pltpu.async_copy(ab_hbm.at[idx_s[p]], gb[p], sgb[p])

  def wait_gathers_c(p):
    pltpu.make_async_copy(ab_hbm.at[idx_d[p]], ga[p], sga[p]).wait()
    pltpu.make_async_copy(ab_hbm.at[idx_s[p]], gb[p], sgb[p]).wait()
    pltpu.make_async_copy(c_hbm.at[pl.ds(0, CHUNK // 2)], cb[p],
                          scc[p]).wait()

  def wait_write(p):
    pltpu.make_async_copy(yb[p], y_hbm.at[pl.ds(0, CHUNK // 2)],
                          sw[p]).wait()

  def compute_write(r, p):
    gap = ga[p]
    gbp = gb[p]
    cbp = cb[p]
    ybp = yb[p]

    def row_body(rr, acc):
      acc = list(acc)
      for par in range(2):
        j = 2 * rr + par
        for g in range(4):
          sl = pl.ds(g * 16, 16)
          slb = pl.ds(D + g * 16, 16)
          slc = pl.ds(par * D + g * 16, 16)
          v = gap[j, sl] + gbp[j, slb] + cbp[rr, slc]
          ybp[rr, slc] = v
          acc[g] = acc[g] + v
          acc[4 + g] = acc[4 + g] + v * v
      return tuple(acc)

    z = jnp.zeros((16,), jnp.float32)
    acc = lax.fori_loop(0, CHUNK // 2, row_body,
                        (z, z, z, z, z, z, z, z))
    for g in range(8):
      sl = pl.ds(g * 16, 16)
      sqb[0, sl] = sqb[0, sl] + acc[g]

    cid = r * NW + wid
    pbase = cid * (CHUNK // 2)
    pltpu.async_copy(ybp, y_hbm.at[pl.ds(pbase, CHUNK // 2)], sw[p])

  issue_inputs(0, 0)
  issue_inputs(1, 1)
  wait_idx(0)
  issue_gathers(0)
  wait_idx(1)
  issue_gathers(1)
  wait_gathers_c(0)
  compute_write(0, 0)
  issue_inputs(2, 0)
  wait_idx(0)
  issue_gathers(0)
  wait_gathers_c(1)
  compute_write(1, 1)
  issue_inputs(3, 1)

  def steady(m, carry):
    k0 = 2 * m + 2
    wait_idx(1)
    issue_gathers(1)
    wait_write(0)
    wait_gathers_c(0)
    compute_write(k0, 0)
    issue_inputs(k0 + 2, 0)
    wait_idx(0)
    issue_gathers(0)
    wait_write(1)
    wait_gathers_c(1)
    compute_write(k0 + 1, 1)
    issue_inputs(k0 + 3, 1)
    return carry

  lax.fori_loop(0, (ROUNDS_W - 4) // 2, steady, 0)

  wait_idx(1)
  issue_gathers(1)
  wait_write(0)
  wait_gathers_c(0)
  compute_write(ROUNDS_W - 2, 0)
  wait_write(1)
  wait_gathers_c(1)
  compute_write(ROUNDS_W - 1, 1)
  wait_write(0)
  wait_write(1)

  pltpu.sync_copy(sqb, sq_hbm.at[wid])


def _sc_combine(ab_tbl, cp, dst, src):
  mesh = plsc.VectorSubcoreMesh(core_axis_name="c", subcore_axis_name="s",
                                num_cores=NC, num_subcores=NS)
  f = pl.kernel(
      _sc_combine_body,
      out_type=[
          jax.ShapeDtypeStruct((PCH, 128), jnp.float32),
          jax.ShapeDtypeStruct((NW, 8, 128), jnp.float32),
      ],
      mesh=mesh,
      scratch_types=(
          [pltpu.VMEM((CHUNK,), jnp.int32)] * 4
          + [pltpu.VMEM((CHUNK, 128), jnp.float32)] * 4
          + [pltpu.VMEM((CHUNK // 2, 128), jnp.float32)] * 4
          + [pltpu.VMEM((8, 128), jnp.float32)]
          + [pltpu.SemaphoreType.DMA] * 12
      ),
  )
  return f(ab_tbl, cp, dst, src)


def _sc_scatter_body(y_hbm, dst_hbm, ac_hbm, out_hbm,
                     tbl, idxe, idxo, ybuf, rowsl, rowsr, zb, acb, sem1):
  c = lax.axis_index("c")
  s = lax.axis_index("s")
  lo = c * HALF

  def zrow(j, carry):
    for g in range(8):
      sl = pl.ds(g * 16, 16)
      zb[j, sl] = jnp.zeros((16,), jnp.float32)
      rowsl[j, sl] = jnp.zeros((16,), jnp.float32)
      rowsr[j, sl] = jnp.zeros((16,), jnp.float32)
    return carry
  lax.fori_loop(0, ZCH, zrow, 0)

  def zrow2(j, carry):
    for g in range(8):
      sl = pl.ds(g * 16, 16)
      rowsl[ZCH + j, sl] = jnp.zeros((16,), jnp.float32)
      rowsr[ZCH + j, sl] = jnp.zeros((16,), jnp.float32)
    return carry
  lax.fori_loop(0, SCHUNK - ZCH, zrow2, 0)

  def zcopy(k, carry):
    pltpu.sync_copy(zb, tbl.at[pl.ds((s * ZCOPIES + k) * ZCH, ZCH)])
    return carry
  lax.fori_loop(0, ZCOPIES, zcopy, 0)

  pltpu.sync_copy(ac_hbm, acb)

  plsc.subcore_barrier()

  a_regs = [acb[0, pl.ds(g * 16, 16)] for g in range(4)]
  c_regs = [acb[1, pl.ds(g * 16, 16)] for g in range(4)]
  lane = lax.iota(jnp.int32, 16)

  def round_body(i, carry):
    cid = i * NS + s
    base = cid * SCHUNK
    pbase = cid * (SCHUNK // 2)
    pltpu.sync_copy(dst_hbm.at[pl.ds(base, SCHUNK)], idxe)
    pltpu.async_copy(y_hbm.at[pl.ds(pbase, SCHUNK // 2)], ybuf, sem1).wait()

    for jj in range(SCHUNK // 16):
      sl = pl.ds(jj * 16, 16)
      v = idxe[sl]
      valid = (v >= lo) & (v < lo + HALF)
      local = v - lo
      packed = lax.shift_right_logical(local, 1)
      oddb = lax.bitwise_and(local, 1)
      dump = TBL_P + lane + 16 * jj
      dump2 = dump + 64
      evens = packed + oddb * (dump - packed)
      odds = packed + (1 - oddb) * (dump2 - packed)
      idxe[sl] = jnp.where(valid, evens, dump)
      idxo[sl] = jnp.where(valid, odds, dump2)

    def row_body(r, carry2):
      for par in range(2):
        j = 2 * r + par
        for g in range(4):
          v = ybuf[r, pl.ds(par * D + g * 16, 16)]
          mv = jnp.maximum(v * a_regs[g] + c_regs[g], 0.0)
          rowsl[j, pl.ds(g * 16, 16)] = mv
          rowsr[j, pl.ds(D + g * 16, 16)] = mv
      return carry2
    lax.fori_loop(0, SCHUNK // 2, row_body, 0)

    pltpu.sync_copy(rowsl, tbl.at[idxe], add=True)
    pltpu.sync_copy(rowsr, tbl.at[idxo], add=True)
    return carry

  lax.fori_loop(0, SROUNDS, round_body, 0)

  plsc.subcore_barrier()

  def drain_body(k, carry):
    prb = s * (TBL_PV // NS) + k * DR_P
    pltpu.sync_copy(tbl.at[pl.ds(prb, DR_P)],
                    out_hbm.at[pl.ds(c * TBL_PV + prb, DR_P)])
    return carry

  lax.fori_loop(0, DR_PER_TILE, drain_body, 0)


def _sc_scatter(y_edge, dst, ac):
  mesh = plsc.VectorSubcoreMesh(core_axis_name="c", subcore_axis_name="s",
                                num_cores=NC, num_subcores=NS)
  f = pl.kernel(
      _sc_scatter_body,
      out_type=jax.ShapeDtypeStruct((2 * TBL_PV, 128), jnp.float32),
      mesh=mesh,
      scratch_types=[
          pltpu.VMEM_SHARED((STBL, 128), jnp.float32),
          pltpu.VMEM((SCHUNK,), jnp.int32),
          pltpu.VMEM((SCHUNK,), jnp.int32),
          pltpu.VMEM((SCHUNK // 2, 128), jnp.float32),
          pltpu.VMEM((SCHUNK, 128), jnp.float32),
          pltpu.VMEM((SCHUNK, 128), jnp.float32),
          pltpu.VMEM((ZCH, 128), jnp.float32),
          pltpu.VMEM((8, 128), jnp.float32),
          pltpu.SemaphoreType.DMA,
      ],
  )
  return f(y_edge, dst, ac)


def _sc_combine_jnp(ab_tbl, cp, dst, src):
  a_t = ab_tbl[:, :D]
  b_t = ab_tbl[:, D:]
  c = cp.reshape(EPAD, D)
  y1 = a_t[dst] + b_t[src] + c
  sq = (jnp.zeros((NW, 8, 128), jnp.float32)
        .at[0, 0, :D].set(y1.sum(0))
        .at[0, 0, D:].set((y1 * y1).sum(0)))
  return y1.reshape(PCH, 128), sq


def _sc_scatter_jnp(y2p, dst, ac):
  y = y2p.reshape(EPAD, D)
  a = ac[0, :D]
  c = ac[1, :D]
  m = jnp.maximum(y * a + c, 0.0)
  aggr = jax.ops.segment_sum(m, dst, num_segments=N).reshape(NP, 128)
  out = jnp.zeros((2 * TBL_PV, 128), jnp.float32)
  out = out.at[:HALF // 2].set(aggr[:HALF // 2])
  return out.at[TBL_PV:TBL_PV + HALF // 2].set(aggr[HALF // 2:])


def _pad_gb(g, b):
  gb = jnp.zeros((8, D), jnp.float32)
  return gb.at[0, :].set(g).at[1, :].set(b)


def kernel(x, edge_attr, prev_h, params, edge_index):
  pad = jnp.full((EPAD - E,), N, jnp.int32)
  src = jnp.concatenate([edge_index[0], pad])
  dst = jnp.concatenate([edge_index[1], pad])

  xp = x.reshape(NP, 2 * IN_DIM)
  hprevp = prev_h.reshape(NP, 128)
  eap = jnp.zeros((PCH, 2 * ED), jnp.float32).at[:EP].set(
      edge_attr.reshape(EP, 2 * ED))

  b_in_hist = params['b_in'] + params['b_hist']
  b128 = jnp.tile(b_in_hist, 2)[None, :]

  hp = _input_proj(xp, hprevp, _bd2(params['W_in']), _bd2(params['W_hist']),
                   b128)

  for l in range(L):
    w1 = params['msg_W1'][l]
    w1a, w1b, w1c = w1[:D], w1[D:2 * D], w1[2 * D:]

    wab = jnp.concatenate([w1a, w1b], axis=1)
    w_e = jnp.zeros((128, 128), jnp.float32).at[:D, :].set(wab)
    w_o = jnp.zeros((128, 128), jnp.float32).at[D:, :].set(wab)

    ab_tbl = _precompute_tbl(hp, w_e, w_o)
    tblp = jnp.zeros((N + 8, 128), jnp.float32).at[:N].set(ab_tbl)
    cp = _edge_c(eap, _bd2(w1c))

    y1, sq1 = _sc_combine(tblp, cp, dst, src)

    ac1 = _finalize(sq1,
                    _pad_gb(params['msg_g1'][l], params['msg_be1'][l]), E)
    y2p, sq2 = _bn_mm_stats(y1, ac1,
                            _bd2(params['msg_W2'][l]), PCH, RB_E,
                            valid_rows=EP)
    ac2 = _finalize_sq(sq2,
                       _pad_gb(params['msg_g2'][l], params['msg_be2'][l]), E)

    slab = _sc_scatter(y2p, dst, ac2)
    aggrp = jnp.concatenate(
        [slab[:HALF // 2], slab[TBL_PV:TBL_PV + HALF // 2]], axis=0)

    u1 = params['upd_W1'][l]
    y3p, sq3 = _update1(hp, aggrp, _bd2(u1[:D]), _bd2(u1[D:]))
    ac3 = _finalize_sq(sq3,
                       _pad_gb(params['upd_g1'][l], params['upd_be1'][l]), N)
    y4p, sq4 = _bn_mm_stats(y3p, ac3, _bd2(params['upd_W2'][l]), NP, RB_N)
    ac4 = _finalize_sq(sq4,
                       _pad_gb(params['upd_g2'][l], params['upd_be2'][l]), N)
    hp = _resid(hp, y4p, ac4)

  wn_bd = _bd2(params['Wn'])
  nb128 = jnp.tile(params['bn'], 2)[None, :]
  node_out, hs = _heads(hp, wn_bd, nb128)

  gp = jnp.zeros((8, D), jnp.float32)
  gp = gp.at[0, :].set(params['Wg'][:, 0]).at[1, 0].set(params['bg'][0])
  graph_out = _graph_head(hs, gp)

  return (graph_out, node_out, hp.reshape(N, D))


# speedup vs baseline: 1.8761x; 1.2889x over previous
import functools

import jax
import jax.numpy as jnp
from jax import lax
from jax.experimental import pallas as pl
from jax.experimental.pallas import tpu as pltpu
from jax.experimental.pallas import tpu_sc as plsc

N = 50000
E = 800000
D = 64
ED = 16
IN_DIM = 128
L = 4

NC = 2
NS = 16
NW = NC * NS

CHUNK = 128
EPAD = 6272 * CHUNK
PCH = EPAD // 2
NCHUNK = EPAD // CHUNK
ROUNDS_W = NCHUNK // NW

HALF = N // 2
TBL_P = HALF // 2
TBL_PV = 12544
STBL = 12800
ZCH = 16
ZCOPIES = STBL // NS // ZCH
DR_P = 16
DR_PER_TILE = TBL_PV // NS // DR_P

SCHUNK = 64
SNCHUNK = EPAD // SCHUNK
SROUNDS = SNCHUNK // NS

RB_N = 1000
RB_E = 3136
EP = E // 2
NP = N // 2

_EPS = 1e-5


def _bd2(w):
  k, m = w.shape
  out = jnp.zeros((2 * k, 2 * m), w.dtype)
  out = out.at[:k, :m].set(w)
  out = out.at[k:, m:].set(w)
  return out


def _k_input(x_ref, hp_ref, wx_ref, wh_ref, b_ref, o_ref):
  acc = jnp.dot(x_ref[...], wx_ref[...], preferred_element_type=jnp.float32)
  acc = acc + jnp.dot(hp_ref[...], wh_ref[...],
                      preferred_element_type=jnp.float32)
  o_ref[...] = jnp.maximum(acc + b_ref[...], 0.0)


def _input_proj(xp, hprevp, wx_bd, wh_bd, b128):
  grid = NP // RB_N
  return pl.pallas_call(
      _k_input,
      grid=(grid,),
      in_specs=[
          pl.BlockSpec((RB_N, 2 * IN_DIM), lambda i: (i, 0)),
          pl.BlockSpec((RB_N, 128), lambda i: (i, 0)),
          pl.BlockSpec((2 * IN_DIM, 128), lambda i: (0, 0)),
          pl.BlockSpec((128, 128), lambda i: (0, 0)),
          pl.BlockSpec((1, 128), lambda i: (0, 0)),
      ],
      out_specs=pl.BlockSpec((RB_N, 128), lambda i: (i, 0)),
      out_shape=jax.ShapeDtypeStruct((NP, 128), jnp.float32),
  )(xp, hprevp, wx_bd, wh_bd, b128)


def _k_ab_tbl(h_ref, we_ref, wo_ref, o_ref):
  h = h_ref[...]
  ev = jnp.dot(h, we_ref[...], preferred_element_type=jnp.float32)
  od = jnp.dot(h, wo_ref[...], preferred_element_type=jnp.float32)
  rb = h.shape[0]
  o_ref[...] = jnp.stack([ev, od], axis=1).reshape(2 * rb, 128)


def _precompute_tbl(hp, w_e, w_o):
  grid = NP // RB_N
  return pl.pallas_call(
      _k_ab_tbl,
      grid=(grid,),
      in_specs=[
          pl.BlockSpec((RB_N, 128), lambda i: (i, 0)),
          pl.BlockSpec((128, 128), lambda i: (0, 0)),
          pl.BlockSpec((128, 128), lambda i: (0, 0)),
      ],
      out_specs=pl.BlockSpec((2 * RB_N, 128), lambda i: (i, 0)),
      out_shape=jax.ShapeDtypeStruct((N, 128), jnp.float32),
  )(hp, w_e, w_o)


def _k_mm(x_ref, w_ref, o_ref):
  o_ref[...] = jnp.dot(x_ref[...], w_ref[...],
                       preferred_element_type=jnp.float32)


def _edge_c(eap, wc_bd):
  grid = PCH // RB_E
  return pl.pallas_call(
      _k_mm,
      grid=(grid,),
      in_specs=[
          pl.BlockSpec((RB_E, 2 * ED), lambda i: (i, 0)),
          pl.BlockSpec((2 * ED, 128), lambda i: (0, 0)),
      ],
      out_specs=pl.BlockSpec((RB_E, 128), lambda i: (i, 0)),
      out_shape=jax.ShapeDtypeStruct((PCH, 128), jnp.float32),
  )(eap, wc_bd)


def _k_bn_mm_stats(y_ref, ac_ref, w_ref, o_ref, st_ref, acc_ref, *,
                   steps, rb, valid_rows):
  i = pl.program_id(0)

  @pl.when(i == 0)
  def _():
    acc_ref[...] = jnp.zeros_like(acc_ref)

  a = ac_ref[0:1, :]
  c = ac_ref[1:2, :]
  m = jnp.maximum(y_ref[...] * a + c, 0.0)
  z = jnp.dot(m, w_ref[...], preferred_element_type=jnp.float32)
  o_ref[...] = z
  if valid_rows == steps * rb:
    zm = z
  else:
    row = lax.broadcasted_iota(jnp.int32, (rb, 1), 0) + i * rb
    zm = jnp.where(row < valid_rows, z, 0.0)
  ps = jnp.sum(zm, axis=0)
  qs = jnp.sum(zm * zm, axis=0)
  new0 = acc_ref[0, :] + ps
  new1 = acc_ref[1, :] + qs
  acc_ref[0, :] = new0
  acc_ref[1, :] = new1

  @pl.when(i == steps - 1)
  def _():
    s64 = (new0[:D] + new0[D:])[None, :]
    q64 = (new1[:D] + new1[D:])[None, :]
    st_ref[...] = jnp.concatenate(
        [s64, q64, jnp.zeros((30, D), jnp.float32)], axis=0)


def _bn_mm_stats(yp, ac, w_bd, rows, rb, valid_rows=None):
  steps = rows // rb
  if valid_rows is None:
    valid_rows = rows
  return pl.pallas_call(
      functools.partial(_k_bn_mm_stats, steps=steps, rb=rb,
                        valid_rows=valid_rows),
      grid=(steps,),
      in_specs=[
          pl.BlockSpec((rb, 128), lambda i: (i, 0)),
          pl.BlockSpec((8, 128), lambda i: (0, 0)),
          pl.BlockSpec((128, 128), lambda i: (0, 0)),
      ],
      out_specs=[
          pl.BlockSpec((rb, 128), lambda i: (i, 0)),
          pl.BlockSpec((32, D), lambda i: (0, 0)),
      ],
      out_shape=[
          jax.ShapeDtypeStruct((rows, 128), jnp.float32),
          jax.ShapeDtypeStruct((32, D), jnp.float32),
      ],
      scratch_shapes=[pltpu.VMEM((8, 128), jnp.float32)],
  )(yp, ac, w_bd)


def _k_update1(h_ref, g_ref, wa_ref, wb_ref, o_ref, st_ref, acc_ref, *, steps):
  i = pl.program_id(0)

  @pl.when(i == 0)
  def _():
    acc_ref[...] = jnp.zeros_like(acc_ref)

  z = jnp.dot(h_ref[...], wa_ref[...], preferred_element_type=jnp.float32)
  z = z + jnp.dot(g_ref[...], wb_ref[...], preferred_element_type=jnp.float32)
  o_ref[...] = z
  new0 = acc_ref[0, :] + jnp.sum(z, axis=0)
  new1 = acc_ref[1, :] + jnp.sum(z * z, axis=0)
  acc_ref[0, :] = new0
  acc_ref[1, :] = new1

  @pl.when(i == steps - 1)
  def _():
    s64 = (new0[:D] + new0[D:])[None, :]
    q64 = (new1[:D] + new1[D:])[None, :]
    st_ref[...] = jnp.concatenate(
        [s64, q64, jnp.zeros((30, D), jnp.float32)], axis=0)


def _update1(hp, aggrp, wa_bd, wb_bd):
  steps = NP // RB_N
  return pl.pallas_call(
      functools.partial(_k_update1, steps=steps),
      grid=(steps,),
      in_specs=[
          pl.BlockSpec((RB_N, 128), lambda i: (i, 0)),
          pl.BlockSpec((RB_N, 128), lambda i: (i, 0)),
          pl.BlockSpec((128, 128), lambda i: (0, 0)),
          pl.BlockSpec((128, 128), lambda i: (0, 0)),
      ],
      out_specs=[
          pl.BlockSpec((RB_N, 128), lambda i: (i, 0)),
          pl.BlockSpec((32, D), lambda i: (0, 0)),
      ],
      out_shape=[
          jax.ShapeDtypeStruct((NP, 128), jnp.float32),
          jax.ShapeDtypeStruct((32, D), jnp.float32),
      ],
      scratch_shapes=[pltpu.VMEM((8, 128), jnp.float32)],
  )(hp, aggrp, wa_bd, wb_bd)


def _k_finalize(sq_ref, gb_ref, o_ref, *, count):
  t = jnp.sum(sq_ref[...].reshape(NW * 8, 128), axis=0)
  s = t[:D]
  q = t[D:]
  mu = s / count
  var = q / count - mu * mu
  a = gb_ref[0, :] * lax.rsqrt(var + _EPS)
  c = gb_ref[1, :] - a * mu
  a128 = jnp.concatenate([a, a])[None, :]
  c128 = jnp.concatenate([c, c])[None, :]
  o_ref[...] = jnp.concatenate(
      [a128, c128, jnp.zeros((6, 128), jnp.float32)], axis=0)


def _finalize(sq, gb, count):
  return pl.pallas_call(
      functools.partial(_k_finalize, count=float(count)),
      in_specs=[
          pl.BlockSpec((NW, 8, 128), lambda: (0, 0, 0)),
          pl.BlockSpec((8, D), lambda: (0, 0)),
      ],
      out_specs=pl.BlockSpec((8, 128), lambda: (0, 0)),
      out_shape=jax.ShapeDtypeStruct((8, 128), jnp.float32),
  )(sq, gb)


def _k_finalize_sq(sq_ref, gb_ref, o_ref, *, count):
  s = sq_ref[0, :]
  q = sq_ref[1, :]
  mu = s / count
  var = q / count - mu * mu
  a = gb_ref[0, :] * lax.rsqrt(var + _EPS)
  c = gb_ref[1, :] - a * mu
  a128 = jnp.concatenate([a, a])[None, :]
  c128 = jnp.concatenate([c, c])[None, :]
  o_ref[...] = jnp.concatenate(
      [a128, c128, jnp.zeros((6, 128), jnp.float32)], axis=0)


def _finalize_sq(sq, gb, count):
  return pl.pallas_call(
      functools.partial(_k_finalize_sq, count=float(count)),
      in_specs=[
          pl.BlockSpec((32, D), lambda: (0, 0)),
          pl.BlockSpec((8, D), lambda: (0, 0)),
      ],
      out_specs=pl.BlockSpec((8, 128), lambda: (0, 0)),
      out_shape=jax.ShapeDtypeStruct((8, 128), jnp.float32),
  )(sq, gb)


def _k_resid(h_ref, y_ref, ac_ref, o_ref):
  a = ac_ref[0:1, :]
  c = ac_ref[1:2, :]
  o_ref[...] = h_ref[...] + jnp.maximum(y_ref[...] * a + c, 0.0)


def _resid(hp, yp, ac):
  grid = NP // RB_N
  return pl.pallas_call(
      _k_resid,
      grid=(grid,),
      in_specs=[
          pl.BlockSpec((RB_N, 128), lambda i: (i, 0)),
          pl.BlockSpec((RB_N, 128), lambda i: (i, 0)),
          pl.BlockSpec((8, 128), lambda i: (0, 0)),
      ],
      out_specs=pl.BlockSpec((RB_N, 128), lambda i: (i, 0)),
      out_shape=jax.ShapeDtypeStruct((NP, 128), jnp.float32),
  )(hp, yp, ac)


def _k_heads(h_ref, wn_ref, no_ref, hs_ref, acc_ref, *, steps):
  i = pl.program_id(0)

  @pl.when(i == 0)
  def _():
    acc_ref[...] = jnp.zeros_like(acc_ref)

  h = h_ref[...]
  no_ref[...] = jnp.dot(h, wn_ref[...], preferred_element_type=jnp.float32)
  new0 = acc_ref[0, :] + jnp.sum(h, axis=0)
  acc_ref[0, :] = new0

  @pl.when(i == steps - 1)
  def _():
    s64 = (new0[:D] + new0[D:])[None, :]
    hs_ref[...] = jnp.concatenate(
        [s64, jnp.zeros((31, D), jnp.float32)], axis=0)


def _heads(hp, wn_bd, nb128):
  steps = NP // RB_N
  nop, hs = pl.pallas_call(
      functools.partial(_k_heads, steps=steps),
      grid=(steps,),
      in_specs=[
          pl.BlockSpec((RB_N, 128), lambda i: (i, 0)),
          pl.BlockSpec((128, 2 * 5), lambda i: (0, 0)),
      ],
      out_specs=[
          pl.BlockSpec((RB_N, 2 * 5), lambda i: (i, 0)),
          pl.BlockSpec((32, D), lambda i: (0, 0)),
      ],
      out_shape=[
          jax.ShapeDtypeStruct((NP, 2 * 5), jnp.float32),
          jax.ShapeDtypeStruct((32, D), jnp.float32),
      ],
      scratch_shapes=[pltpu.VMEM((8, 128), jnp.float32)],
  )(hp, wn_bd)
  nop = (nop + nb128).reshape(N, 5)
  return nop, hs


def _k_graph(hs_ref, gp_ref, o_ref):
  hmean = jnp.sum(hs_ref[...], axis=0) / float(N)
  g = jnp.sum(hmean * gp_ref[0, :]) + gp_ref[1, 0]
  o_ref[...] = jnp.full((8, 128), g, jnp.float32)


def _graph_head(hs, gp):
  out = pl.pallas_call(
      _k_graph,
      in_specs=[
          pl.BlockSpec((32, D), lambda: (0, 0)),
          pl.BlockSpec((8, D), lambda: (0, 0)),
      ],
      out_specs=pl.BlockSpec((8, 128), lambda: (0, 0)),
      out_shape=jax.ShapeDtypeStruct((8, 128), jnp.float32),
  )(hs, gp)
  return out[0, 0:1]


def _sc_combine_body(ab_hbm, c_hbm, dst_hbm, src_hbm,
                     y_hbm, sq_hbm,
                     idx_d0, idx_d1, idx_s0, idx_s1,
                     ga0, ga1, gb0, gb1, cb0, cb1, yb0, yb1, sqb,
                     sid0, sid1, sis0, sis1, sga0, sga1, sgb0, sgb1,
                     scc0, scc1, sw0, sw1):
  wid = lax.axis_index("s") * NC + lax.axis_index("c")
  idx_d = [idx_d0, idx_d1]
  idx_s = [idx_s0, idx_s1]
  ga = [ga0, ga1]
  gb = [gb0, gb1]
  cb = [cb0, cb1]
  yb = [yb0, yb1]
  sid = [sid0, sid1]
  sis = [sis0, sis1]
  sga = [sga0, sga1]
  sgb = [sgb0, sgb1]
  scc = [scc0, scc1]
  sw = [sw0, sw1]

  for r in range(8):
    for g in range(8):
      sqb[r, pl.ds(g * 16, 16)] = jnp.zeros((16,), jnp.float32)

  def issue_inputs(r, p):
    cid = r * NW + wid
    base = cid * CHUNK
    pbase = cid * (CHUNK // 2)
    pltpu.async_copy(dst_hbm.at[pl.ds(base, CHUNK)], idx_d[p], sid[p])
    pltpu.async_copy(src_hbm.at[pl.ds(base, CHUNK)], idx_s[p], sis[p])
    pltpu.async_copy(c_hbm.at[pl.ds(pbase, CHUNK // 2)], cb[p], scc[p])

  def wait_idx(p):
    pltpu.make_async_copy(dst_hbm.at[pl.ds(0, CHUNK)], idx_d[p],
                          sid[p]).wait()
    pltpu.make_async_copy(src_hbm.at[pl.ds(0, CHUNK)], idx_s[p],
                          sis[p]).wait()

  def issue_gathers(p):
    pltpu.async_copy(ab_hbm.at[idx_d[p]], ga[p], sga[p])
    pltpu.async_copy(ab_hbm.at[idx_s[p]], gb[p], sgb[p])

  def wait_gathers_c(p):
    pltpu.make_async_copy(ab_hbm.at[idx_d[p]], ga[p], sga[p]).wait()
    pltpu.make_async_copy(ab_hbm.at[idx_s[p]], gb[p], sgb[p]).wait()
    pltpu.make_async_copy(c_hbm.at[pl.ds(0, CHUNK // 2)], cb[p],
                          scc[p]).wait()

  def wait_write(p):
    pltpu.make_async_copy(yb[p], y_hbm.at[pl.ds(0, CHUNK // 2)],
                          sw[p]).wait()

  def compute_write(r, p):
    gap = ga[p]
    gbp = gb[p]
    cbp = cb[p]
    ybp = yb[p]

    def row_body(rr, acc):
      acc = list(acc)
      for par in range(2):
        j = 2 * rr + par
        for g in range(4):
          sl = pl.ds(g * 16, 16)
          slb = pl.ds(D + g * 16, 16)
          slc = pl.ds(par * D + g * 16, 16)
          v = gap[j, sl] + gbp[j, slb] + cbp[rr, slc]
          ybp[rr, slc] = v
          acc[g] = acc[g] + v
          acc[4 + g] = acc[4 + g] + v * v
      return tuple(acc)

    z = jnp.zeros((16,), jnp.float32)
    acc = lax.fori_loop(0, CHUNK // 2, row_body,
                        (z, z, z, z, z, z, z, z))
    for g in range(8):
      sl = pl.ds(g * 16, 16)
      sqb[0, sl] = sqb[0, sl] + acc[g]

    cid = r * NW + wid
    pbase = cid * (CHUNK // 2)
    pltpu.async_copy(ybp, y_hbm.at[pl.ds(pbase, CHUNK // 2)], sw[p])

  issue_inputs(0, 0)
  issue_inputs(1, 1)
  wait_idx(0)
  issue_gathers(0)
  wait_idx(1)
  issue_gathers(1)
  wait_gathers_c(0)
  compute_write(0, 0)
  issue_inputs(2, 0)
  wait_idx(0)
  issue_gathers(0)
  wait_gathers_c(1)
  compute_write(1, 1)
  issue_inputs(3, 1)

  def steady(m, carry):
    k0 = 2 * m + 2
    wait_idx(1)
    issue_gathers(1)
    wait_write(0)
    wait_gathers_c(0)
    compute_write(k0, 0)
    issue_inputs(k0 + 2, 0)
    wait_idx(0)
    issue_gathers(0)
    wait_write(1)
    wait_gathers_c(1)
    compute_write(k0 + 1, 1)
    issue_inputs(k0 + 3, 1)
    return carry

  lax.fori_loop(0, (ROUNDS_W - 4) // 2, steady, 0)

  wait_idx(1)
  issue_gathers(1)
  wait_write(0)
  wait_gathers_c(0)
  compute_write(ROUNDS_W - 2, 0)
  wait_write(1)
  wait_gathers_c(1)
  compute_write(ROUNDS_W - 1, 1)
  wait_write(0)
  wait_write(1)

  pltpu.sync_copy(sqb, sq_hbm.at[wid])


def _sc_combine(ab_tbl, cp, dst, src):
  mesh = plsc.VectorSubcoreMesh(core_axis_name="c", subcore_axis_name="s",
                                num_cores=NC, num_subcores=NS)
  f = pl.kernel(
      _sc_combine_body,
      out_type=[
          jax.ShapeDtypeStruct((PCH, 128), jnp.float32),
          jax.ShapeDtypeStruct((NW, 8, 128), jnp.float32),
      ],
      mesh=mesh,
      scratch_types=(
          [pltpu.VMEM((CHUNK,), jnp.int32)] * 4
          + [pltpu.VMEM((CHUNK, 128), jnp.float32)] * 4
          + [pltpu.VMEM((CHUNK // 2, 128), jnp.float32)] * 4
          + [pltpu.VMEM((8, 128), jnp.float32)]
          + [pltpu.SemaphoreType.DMA] * 12
      ),
  )
  return f(ab_tbl, cp, dst, src)


def _sc_scatter_body(y_hbm, dst_hbm, ac_hbm, out_hbm,
                     tbl, idxe0, idxe1, idxo0, idxo1, ybuf0, ybuf1,
                     rowsl, rowsr, zb, acb,
                     sld0, sld1, sy0, sy1, ssc):
  c = lax.axis_index("c")
  s = lax.axis_index("s")
  lo = c * HALF
  idxe = [idxe0, idxe1]
  idxo = [idxo0, idxo1]
  ybuf = [ybuf0, ybuf1]
  sld = [sld0, sld1]
  sy = [sy0, sy1]

  def zrow(j, carry):
    for g in range(8):
      sl = pl.ds(g * 16, 16)
      zb[j, sl] = jnp.zeros((16,), jnp.float32)
    return carry
  lax.fori_loop(0, ZCH, zrow, 0)

  def zrow2(j, carry):
    for g in range(8):
      sl = pl.ds(g * 16, 16)
      rowsl[j, sl] = jnp.zeros((16,), jnp.float32)
      rowsr[j, sl] = jnp.zeros((16,), jnp.float32)
    return carry
  lax.fori_loop(0, SCHUNK, zrow2, 0)

  def zcopy(k, carry):
    pltpu.sync_copy(zb, tbl.at[pl.ds((s * ZCOPIES + k) * ZCH, ZCH)])
    return carry
  lax.fori_loop(0, ZCOPIES, zcopy, 0)

  pltpu.sync_copy(ac_hbm, acb)

  plsc.subcore_barrier()

  a_regs = [acb[0, pl.ds(g * 16, 16)] for g in range(4)]
  c_regs = [acb[1, pl.ds(g * 16, 16)] for g in range(4)]
  lane = lax.iota(jnp.int32, 16)

  def issue_loads(i, p):
    cid = i * NS + s
    base = cid * SCHUNK
    pbase = cid * (SCHUNK // 2)
    pltpu.async_copy(dst_hbm.at[pl.ds(base, SCHUNK)], idxe[p], sld[p])
    pltpu.async_copy(y_hbm.at[pl.ds(pbase, SCHUNK // 2)], ybuf[p], sy[p])

  def wait_loads(p):
    pltpu.make_async_copy(dst_hbm.at[pl.ds(0, SCHUNK)], idxe[p],
                          sld[p]).wait()
    pltpu.make_async_copy(y_hbm.at[pl.ds(0, SCHUNK // 2)], ybuf[p],
                          sy[p]).wait()

  def wait_scatters(p):
    pltpu.make_async_copy(rowsl, tbl.at[idxe[p]], ssc).wait()
    pltpu.make_async_copy(rowsr, tbl.at[idxo[p]], ssc).wait()

  def compute_scatter(p):
    for jj in range(SCHUNK // 16):
      sl = pl.ds(jj * 16, 16)
      v = idxe[p][sl]
      valid = (v >= lo) & (v < lo + HALF)
      local = v - lo
      packed = lax.shift_right_logical(local, 1)
      oddb = lax.bitwise_and(local, 1)
      dump = TBL_P + lane + 16 * jj
      dump2 = dump + 64
      evens = packed + oddb * (dump - packed)
      odds = packed + (1 - oddb) * (dump2 - packed)
      idxe[p][sl] = jnp.where(valid, evens, dump)
      idxo[p][sl] = jnp.where(valid, odds, dump2)

    def row_body(r, carry2):
      for par in range(2):
        j = 2 * r + par
        for g in range(4):
          v = ybuf[p][r, pl.ds(par * D + g * 16, 16)]
          mv = jnp.maximum(v * a_regs[g] + c_regs[g], 0.0)
          rowsl[j, pl.ds(g * 16, 16)] = mv
          rowsr[j, pl.ds(D + g * 16, 16)] = mv
      return carry2
    lax.fori_loop(0, SCHUNK // 2, row_body, 0)

    pltpu.async_copy(rowsl, tbl.at[idxe[p]], ssc, add=True)
    pltpu.async_copy(rowsr, tbl.at[idxo[p]], ssc, add=True)

  issue_loads(0, 0)
  wait_loads(0)
  issue_loads(1, 1)
  compute_scatter(0)

  def steady(m, carry):
    i0 = 2 * m + 1
    wait_loads(1)
    wait_scatters(0)
    issue_loads(i0 + 1, 0)
    compute_scatter(1)
    wait_loads(0)
    wait_scatters(1)
    issue_loads(i0 + 2, 1)
    compute_scatter(0)
    return carry

  lax.fori_loop(0, (SROUNDS - 2) // 2, steady, 0)

  wait_loads(1)
  wait_scatters(0)
  compute_scatter(1)
  wait_scatters(1)

  plsc.subcore_barrier()

  def drain_body(k, carry):
    prb = s * (TBL_PV // NS) + k * DR_P
    pltpu.sync_copy(tbl.at[pl.ds(prb, DR_P)],
                    out_hbm.at[pl.ds(c * TBL_PV + prb, DR_P)])
    return carry

  lax.fori_loop(0, DR_PER_TILE, drain_body, 0)


def _sc_scatter(y_edge, dst, ac):
  mesh = plsc.VectorSubcoreMesh(core_axis_name="c", subcore_axis_name="s",
                                num_cores=NC, num_subcores=NS)
  f = pl.kernel(
      _sc_scatter_body,
      out_type=jax.ShapeDtypeStruct((2 * TBL_PV, 128), jnp.float32),
      mesh=mesh,
      scratch_types=(
          [pltpu.VMEM_SHARED((STBL, 128), jnp.float32)]
          + [pltpu.VMEM((SCHUNK,), jnp.int32)] * 4
          + [pltpu.VMEM((SCHUNK // 2, 128), jnp.float32)] * 2
          + [pltpu.VMEM((SCHUNK, 128), jnp.float32)] * 2
          + [pltpu.VMEM((ZCH, 128), jnp.float32)]
          + [pltpu.VMEM((8, 128), jnp.float32)]
          + [pltpu.SemaphoreType.DMA] * 5
      ),
  )
  return f(y_edge, dst, ac)


def _sc_combine_jnp(ab_tbl, cp, dst, src):
  a_t = ab_tbl[:, :D]
  b_t = ab_tbl[:, D:]
  c = cp.reshape(EPAD, D)
  y1 = a_t[dst] + b_t[src] + c
  sq = (jnp.zeros((NW, 8, 128), jnp.float32)
        .at[0, 0, :D].set(y1.sum(0))
        .at[0, 0, D:].set((y1 * y1).sum(0)))
  return y1.reshape(PCH, 128), sq


def _sc_scatter_jnp(y2p, dst, ac):
  y = y2p.reshape(EPAD, D)
  a = ac[0, :D]
  c = ac[1, :D]
  m = jnp.maximum(y * a + c, 0.0)
  aggr = jax.ops.segment_sum(m, dst, num_segments=N).reshape(NP, 128)
  out = jnp.zeros((2 * TBL_PV, 128), jnp.float32)
  out = out.at[:HALF // 2].set(aggr[:HALF // 2])
  return out.at[TBL_PV:TBL_PV + HALF // 2].set(aggr[HALF // 2:])


def _pad_gb(g, b):
  gb = jnp.zeros((8, D), jnp.float32)
  return gb.at[0, :].set(g).at[1, :].set(b)


def kernel(x, edge_attr, prev_h, params, edge_index):
  pad = jnp.full((EPAD - E,), N, jnp.int32)
  src = jnp.concatenate([edge_index[0], pad])
  dst = jnp.concatenate([edge_index[1], pad])

  xp = x.reshape(NP, 2 * IN_DIM)
  hprevp = prev_h.reshape(NP, 128)
  eap = jnp.zeros((PCH, 2 * ED), jnp.float32).at[:EP].set(
      edge_attr.reshape(EP, 2 * ED))

  b_in_hist = params['b_in'] + params['b_hist']
  b128 = jnp.tile(b_in_hist, 2)[None, :]

  hp = _input_proj(xp, hprevp, _bd2(params['W_in']), _bd2(params['W_hist']),
                   b128)

  for l in range(L):
    w1 = params['msg_W1'][l]
    w1a, w1b, w1c = w1[:D], w1[D:2 * D], w1[2 * D:]

    wab = jnp.concatenate([w1a, w1b], axis=1)
    w_e = jnp.zeros((128, 128), jnp.float32).at[:D, :].set(wab)
    w_o = jnp.zeros((128, 128), jnp.float32).at[D:, :].set(wab)

    ab_tbl = _precompute_tbl(hp, w_e, w_o)
    tblp = jnp.zeros((N + 8, 128), jnp.float32).at[:N].set(ab_tbl)
    cp = _edge_c(eap, _bd2(w1c))

    y1, sq1 = _sc_combine(tblp, cp, dst, src)

    ac1 = _finalize(sq1,
                    _pad_gb(params['msg_g1'][l], params['msg_be1'][l]), E)
    y2p, sq2 = _bn_mm_stats(y1, ac1,
                            _bd2(params['msg_W2'][l]), PCH, RB_E,
                            valid_rows=EP)
    ac2 = _finalize_sq(sq2,
                       _pad_gb(params['msg_g2'][l], params['msg_be2'][l]), E)

    slab = _sc_scatter(y2p, dst, ac2)
    aggrp = jnp.concatenate(
        [slab[:HALF // 2], slab[TBL_PV:TBL_PV + HALF // 2]], axis=0)

    u1 = params['upd_W1'][l]
    y3p, sq3 = _update1(hp, aggrp, _bd2(u1[:D]), _bd2(u1[D:]))
    ac3 = _finalize_sq(sq3,
                       _pad_gb(params['upd_g1'][l], params['upd_be1'][l]), N)
    y4p, sq4 = _bn_mm_stats(y3p, ac3, _bd2(params['upd_W2'][l]), NP, RB_N)
    ac4 = _finalize_sq(sq4,
                       _pad_gb(params['upd_g2'][l], params['upd_be2'][l]), N)
    hp = _resid(hp, y4p, ac4)

  wn_bd = _bd2(params['Wn'])
  nb128 = jnp.tile(params['bn'], 2)[None, :]
  node_out, hs = _heads(hp, wn_bd, nb128)

  gp = jnp.zeros((8, D), jnp.float32)
  gp = gp.at[0, :].set(params['Wg'][:, 0]).at[1, 0].set(params['bg'][0])
  graph_out = _graph_head(hs, gp)

  return (graph_out, node_out, hp.reshape(N, D))


# speedup vs baseline: 1.8761x; 1.0000x over previous
import functools

import jax
import jax.numpy as jnp
from jax import lax
from jax.experimental import pallas as pl
from jax.experimental.pallas import tpu as pltpu
from jax.experimental.pallas import tpu_sc as plsc

N = 50000
E = 800000
D = 64
ED = 16
IN_DIM = 128
L = 4

NC = 2
NS = 16
NW = NC * NS

CHUNK = 128
EPAD = 6272 * CHUNK
PCH = EPAD // 2
NCHUNK = EPAD // CHUNK
ROUNDS_W = NCHUNK // NW

HALF = N // 2
TBL_P = HALF // 2
TBL_PV = 12544
STBL = 12800
ZCH = 16
ZCOPIES = STBL // NS // ZCH
DR_P = 16
DR_PER_TILE = TBL_PV // NS // DR_P

SCHUNK = 64
SNCHUNK = EPAD // SCHUNK
SROUNDS = SNCHUNK // NS

RB_N = 1000
RB_E = 3136
EP = E // 2
NP = N // 2

_EPS = 1e-5


def _bd2(w):
  k, m = w.shape
  out = jnp.zeros((2 * k, 2 * m), w.dtype)
  out = out.at[:k, :m].set(w)
  out = out.at[k:, m:].set(w)
  return out


def _k_input(x_ref, hp_ref, wx_ref, wh_ref, b_ref, o_ref):
  acc = jnp.dot(x_ref[...], wx_ref[...], preferred_element_type=jnp.float32)
  acc = acc + jnp.dot(hp_ref[...], wh_ref[...],
                      preferred_element_type=jnp.float32)
  o_ref[...] = jnp.maximum(acc + b_ref[...], 0.0)


def _input_proj(xp, hprevp, wx_bd, wh_bd, b128):
  grid = NP // RB_N
  return pl.pallas_call(
      _k_input,
      grid=(grid,),
      in_specs=[
          pl.BlockSpec((RB_N, 2 * IN_DIM), lambda i: (i, 0)),
          pl.BlockSpec((RB_N, 128), lambda i: (i, 0)),
          pl.BlockSpec((2 * IN_DIM, 128), lambda i: (0, 0)),
          pl.BlockSpec((128, 128), lambda i: (0, 0)),
          pl.BlockSpec((1, 128), lambda i: (0, 0)),
      ],
      out_specs=pl.BlockSpec((RB_N, 128), lambda i: (i, 0)),
      out_shape=jax.ShapeDtypeStruct((NP, 128), jnp.float32),
  )(xp, hprevp, wx_bd, wh_bd, b128)


def _k_ab_tbl(h_ref, we_ref, wo_ref, o_ref):
  h = h_ref[...]
  ev = jnp.dot(h, we_ref[...], preferred_element_type=jnp.float32)
  od = jnp.dot(h, wo_ref[...], preferred_element_type=jnp.float32)
  rb = h.shape[0]
  o_ref[...] = jnp.stack([ev, od], axis=1).reshape(2 * rb, 128)


def _precompute_tbl(hp, w_e, w_o):
  grid = NP // RB_N
  return pl.pallas_call(
      _k_ab_tbl,
      grid=(grid,),
      in_specs=[
          pl.BlockSpec((RB_N, 128), lambda i: (i, 0)),
          pl.BlockSpec((128, 128), lambda i: (0, 0)),
          pl.BlockSpec((128, 128), lambda i: (0, 0)),
      ],
      out_specs=pl.BlockSpec((2 * RB_N, 128), lambda i: (i, 0)),
      out_shape=jax.ShapeDtypeStruct((N, 128), jnp.float32),
  )(hp, w_e, w_o)


def _k_mm(x_ref, w_ref, o_ref):
  o_ref[...] = jnp.dot(x_ref[...], w_ref[...],
                       preferred_element_type=jnp.float32)


def _edge_c(eap, wc_bd):
  grid = PCH // RB_E
  return pl.pallas_call(
      _k_mm,
      grid=(grid,),
      in_specs=[
          pl.BlockSpec((RB_E, 2 * ED), lambda i: (i, 0)),
          pl.BlockSpec((2 * ED, 128), lambda i: (0, 0)),
      ],
      out_specs=pl.BlockSpec((RB_E, 128), lambda i: (i, 0)),
      out_shape=jax.ShapeDtypeStruct((PCH, 128), jnp.float32),
  )(eap, wc_bd)


def _k_bn_mm_stats(y_ref, ac_ref, w_ref, o_ref, st_ref, acc_ref, *,
                   steps, rb, valid_rows):
  i = pl.program_id(0)

  @pl.when(i == 0)
  def _():
    acc_ref[...] = jnp.zeros_like(acc_ref)

  a = ac_ref[0:1, :]
  c = ac_ref[1:2, :]
  m = jnp.maximum(y_ref[...] * a + c, 0.0)
  z = jnp.dot(m, w_ref[...], preferred_element_type=jnp.float32)
  o_ref[...] = z
  if valid_rows == steps * rb:
    zm = z
  else:
    row = lax.broadcasted_iota(jnp.int32, (rb, 1), 0) + i * rb
    zm = jnp.where(row < valid_rows, z, 0.0)
  ps = jnp.sum(zm, axis=0)
  qs = jnp.sum(zm * zm, axis=0)
  new0 = acc_ref[0, :] + ps
  new1 = acc_ref[1, :] + qs
  acc_ref[0, :] = new0
  acc_ref[1, :] = new1

  @pl.when(i == steps - 1)
  def _():
    s64 = (new0[:D] + new0[D:])[None, :]
    q64 = (new1[:D] + new1[D:])[None, :]
    st_ref[...] = jnp.concatenate(
        [s64, q64, jnp.zeros((30, D), jnp.float32)], axis=0)


def _bn_mm_stats(yp, ac, w_bd, rows, rb, valid_rows=None):
  steps = rows // rb
  if valid_rows is None:
    valid_rows = rows
  return pl.pallas_call(
      functools.partial(_k_bn_mm_stats, steps=steps, rb=rb,
                        valid_rows=valid_rows),
      grid=(steps,),
      in_specs=[
          pl.BlockSpec((rb, 128), lambda i: (i, 0)),
          pl.BlockSpec((8, 128), lambda i: (0, 0)),
          pl.BlockSpec((128, 128), lambda i: (0, 0)),
      ],
      out_specs=[
          pl.BlockSpec((rb, 128), lambda i: (i, 0)),
          pl.BlockSpec((32, D), lambda i: (0, 0)),
      ],
      out_shape=[
          jax.ShapeDtypeStruct((rows, 128), jnp.float32),
          jax.ShapeDtypeStruct((32, D), jnp.float32),
      ],
      scratch_shapes=[pltpu.VMEM((8, 128), jnp.float32)],
  )(yp, ac, w_bd)


def _k_update1(h_ref, g_ref, wa_ref, wb_ref, o_ref, st_ref, acc_ref, *, steps):
  i = pl.program_id(0)

  @pl.when(i == 0)
  def _():
    acc_ref[...] = jnp.zeros_like(acc_ref)

  z = jnp.dot(h_ref[...], wa_ref[...], preferred_element_type=jnp.float32)
  z = z + jnp.dot(g_ref[...], wb_ref[...], preferred_element_type=jnp.float32)
  o_ref[...] = z
  new0 = acc_ref[0, :] + jnp.sum(z, axis=0)
  new1 = acc_ref[1, :] + jnp.sum(z * z, axis=0)
  acc_ref[0, :] = new0
  acc_ref[1, :] = new1

  @pl.when(i == steps - 1)
  def _():
    s64 = (new0[:D] + new0[D:])[None, :]
    q64 = (new1[:D] + new1[D:])[None, :]
    st_ref[...] = jnp.concatenate(
        [s64, q64, jnp.zeros((30, D), jnp.float32)], axis=0)


def _update1(hp, aggrp, wa_bd, wb_bd):
  steps = NP // RB_N
  return pl.pallas_call(
      functools.partial(_k_update1, steps=steps),
      grid=(steps,),
      in_specs=[
          pl.BlockSpec((RB_N, 128), lambda i: (i, 0)),
          pl.BlockSpec((RB_N, 128), lambda i: (i, 0)),
          pl.BlockSpec((128, 128), lambda i: (0, 0)),
          pl.BlockSpec((128, 128), lambda i: (0, 0)),
      ],
      out_specs=[
          pl.BlockSpec((RB_N, 128), lambda i: (i, 0)),
          pl.BlockSpec((32, D), lambda i: (0, 0)),
      ],
      out_shape=[
          jax.ShapeDtypeStruct((NP, 128), jnp.float32),
          jax.ShapeDtypeStruct((32, D), jnp.float32),
      ],
      scratch_shapes=[pltpu.VMEM((8, 128), jnp.float32)],
  )(hp, aggrp, wa_bd, wb_bd)


def _k_finalize(sq_ref, gb_ref, o_ref, *, count):
  t = jnp.sum(sq_ref[...].reshape(NW * 8, 128), axis=0)
  s = t[:D]
  q = t[D:]
  mu = s / count
  var = q / count - mu * mu
  a = gb_ref[0, :] * lax.rsqrt(var + _EPS)
  c = gb_ref[1, :] - a * mu
  a128 = jnp.concatenate([a, a])[None, :]
  c128 = jnp.concatenate([c, c])[None, :]
  o_ref[...] = jnp.concatenate(
      [a128, c128, jnp.zeros((6, 128), jnp.float32)], axis=0)


def _finalize(sq, gb, count):
  return pl.pallas_call(
      functools.partial(_k_finalize, count=float(count)),
      in_specs=[
          pl.BlockSpec((NW, 8, 128), lambda: (0, 0, 0)),
          pl.BlockSpec((8, D), lambda: (0, 0)),
      ],
      out_specs=pl.BlockSpec((8, 128), lambda: (0, 0)),
      out_shape=jax.ShapeDtypeStruct((8, 128), jnp.float32),
  )(sq, gb)


def _k_finalize_sq(sq_ref, gb_ref, o_ref, *, count):
  s = sq_ref[0, :]
  q = sq_ref[1, :]
  mu = s / count
  var = q / count - mu * mu
  a = gb_ref[0, :] * lax.rsqrt(var + _EPS)
  c = gb_ref[1, :] - a * mu
  a128 = jnp.concatenate([a, a])[None, :]
  c128 = jnp.concatenate([c, c])[None, :]
  o_ref[...] = jnp.concatenate(
      [a128, c128, jnp.zeros((6, 128), jnp.float32)], axis=0)


def _finalize_sq(sq, gb, count):
  return pl.pallas_call(
      functools.partial(_k_finalize_sq, count=float(count)),
      in_specs=[
          pl.BlockSpec((32, D), lambda: (0, 0)),
          pl.BlockSpec((8, D), lambda: (0, 0)),
      ],
      out_specs=pl.BlockSpec((8, 128), lambda: (0, 0)),
      out_shape=jax.ShapeDtypeStruct((8, 128), jnp.float32),
  )(sq, gb)


def _k_resid(h_ref, y_ref, ac_ref, o_ref):
  a = ac_ref[0:1, :]
  c = ac_ref[1:2, :]
  o_ref[...] = h_ref[...] + jnp.maximum(y_ref[...] * a + c, 0.0)


def _resid(hp, yp, ac):
  grid = NP // RB_N
  return pl.pallas_call(
      _k_resid,
      grid=(grid,),
      in_specs=[
          pl.BlockSpec((RB_N, 128), lambda i: (i, 0)),
          pl.BlockSpec((RB_N, 128), lambda i: (i, 0)),
          pl.BlockSpec((8, 128), lambda i: (0, 0)),
      ],
      out_specs=pl.BlockSpec((RB_N, 128), lambda i: (i, 0)),
      out_shape=jax.ShapeDtypeStruct((NP, 128), jnp.float32),
  )(hp, yp, ac)


def _k_heads(h_ref, wn_ref, no_ref, hs_ref, acc_ref, *, steps):
  i = pl.program_id(0)

  @pl.when(i == 0)
  def _():
    acc_ref[...] = jnp.zeros_like(acc_ref)

  h = h_ref[...]
  no_ref[...] = jnp.dot(h, wn_ref[...], preferred_element_type=jnp.float32)
  new0 = acc_ref[0, :] + jnp.sum(h, axis=0)
  acc_ref[0, :] = new0

  @pl.when(i == steps - 1)
  def _():
    s64 = (new0[:D] + new0[D:])[None, :]
    hs_ref[...] = jnp.concatenate(
        [s64, jnp.zeros((31, D), jnp.float32)], axis=0)


def _heads(hp, wn_bd, nb128):
  steps = NP // RB_N
  nop, hs = pl.pallas_call(
      functools.partial(_k_heads, steps=steps),
      grid=(steps,),
      in_specs=[
          pl.BlockSpec((RB_N, 128), lambda i: (i, 0)),
          pl.BlockSpec((128, 2 * 5), lambda i: (0, 0)),
      ],
      out_specs=[
          pl.BlockSpec((RB_N, 2 * 5), lambda i: (i, 0)),
          pl.BlockSpec((32, D), lambda i: (0, 0)),
      ],
      out_shape=[
          jax.ShapeDtypeStruct((NP, 2 * 5), jnp.float32),
          jax.ShapeDtypeStruct((32, D), jnp.float32),
      ],
      scratch_shapes=[pltpu.VMEM((8, 128), jnp.float32)],
  )(hp, wn_bd)
  nop = (nop + nb128).reshape(N, 5)
  return nop, hs


def _k_graph(hs_ref, gp_ref, o_ref):
  hmean = jnp.sum(hs_ref[...], axis=0) / float(N)
  g = jnp.sum(hmean * gp_ref[0, :]) + gp_ref[1, 0]
  o_ref[...] = jnp.full((8, 128), g, jnp.float32)


def _graph_head(hs, gp):
  out = pl.pallas_call(
      _k_graph,
      in_specs=[
          pl.BlockSpec((32, D), lambda: (0, 0)),
          pl.BlockSpec((8, D), lambda: (0, 0)),
      ],
      out_specs=pl.BlockSpec((8, 128), lambda: (0, 0)),
      out_shape=jax.ShapeDtypeStruct((8, 128), jnp.float32),
  )(hs, gp)
  return out[0, 0:1]


def _sc_combine_body(ab_hbm, c_hbm, dst_hbm, src_hbm,
                     y_hbm, sq_hbm,
                     idx_d0, idx_d1, idx_s0, idx_s1,
                     ga0, ga1, gb0, gb1, cb0, cb1, yb0, yb1, sqb,
                     sid0, sid1, sis0, sis1, sga0, sga1, sgb0, sgb1,
                     scc0, scc1, sw0, sw1):
  wid = lax.axis_index("s") * NC + lax.axis_index("c")
  idx_d = [idx_d0, idx_d1]
  idx_s = [idx_s0, idx_s1]
  ga = [ga0, ga1]
  gb = [gb0, gb1]
  cb = [cb0, cb1]
  yb = [yb0, yb1]
  sid = [sid0, sid1]
  sis = [sis0, sis1]
  sga = [sga0, sga1]
  sgb = [sgb0, sgb1]
  scc = [scc0, scc1]
  sw = [sw0, sw1]

  for r in range(8):
    for g in range(8):
      sqb[r, pl.ds(g * 16, 16)] = jnp.zeros((16,), jnp.float32)

  def issue_inputs(r, p):
    cid = r * NW + wid
    base = cid * CHUNK
    pbase = cid * (CHUNK // 2)
    pltpu.async_copy(dst_hbm.at[pl.ds(base, CHUNK)], idx_d[p], sid[p])
    pltpu.async_copy(src_hbm.at[pl.ds(base, CHUNK)], idx_s[p], sis[p])
    pltpu.async_copy(c_hbm.at[pl.ds(pbase, CHUNK // 2)], cb[p], scc[p])

  def wait_idx(p):
    pltpu.make_async_copy(dst_hbm.at[pl.ds(0, CHUNK)], idx_d[p],
                          sid[p]).wait()
    pltpu.make_async_copy(src_hbm.at[pl.ds(0, CHUNK)], idx_s[p],
                          sis[p]).wait()

  def issue_gathers(p):
    pltpu.async_copy(ab_hbm.at[idx_d[p]], ga[p], sga[p])
    pltpu.async_copy(ab_hbm.at[idx_s[p]], gb[p], sgb[p])

  def wait_gathers_c(p):
    pltpu.make_async_copy(ab_hbm.at[idx_d[p]], ga[p], sga[p]).wait()
    pltpu.make_async_copy(ab_hbm.at[idx_s[p]], gb[p], sgb[p]).wait()
    pltpu.make_async_copy(c_hbm.at[pl.ds(0, CHUNK // 2)], cb[p],
                          scc[p]).wait()

  def wait_write(p):
    pltpu.make_async_copy(yb[p], y_hbm.at[pl.ds(0, CHUNK // 2)],
                          sw[p]).wait()

  def compute_write(r, p):
    gap = ga[p]
    gbp = gb[p]
    cbp = cb[p]
    ybp = yb[p]

    def row_body(rr, acc):
      acc = list(acc)
      for par in range(2):
        j = 2 * rr + par
        for g in range(4):
          sl = pl.ds(g * 16, 16)
          slb = pl.ds(D + g * 16, 16)
          slc = pl.ds(par * D + g * 16, 16)
          v = gap[j, sl] + gbp[j, slb] + cbp[rr, slc]
          ybp[rr, slc] = v
          acc[g] = acc[g] + v
          acc[4 + g] = acc[4 + g] + v * v
      return tuple(acc)

    z = jnp.zeros((16,), jnp.float32)
    acc = lax.fori_loop(0, CHUNK // 2, row_body,
                        (z, z, z, z, z, z, z, z))
    for g in range(8):
      sl = pl.ds(g * 16, 16)
      sqb[0, sl] = sqb[0, sl] + acc[g]

    cid = r * NW + wid
    pbase = cid * (CHUNK // 2)
    pltpu.async_copy(ybp, y_hbm.at[pl.ds(pbase, CHUNK // 2)], sw[p])

  issue_inputs(0, 0)
  issue_inputs(1, 1)
  wait_idx(0)
  issue_gathers(0)
  wait_idx(1)
  issue_gathers(1)
  wait_gathers_c(0)
  compute_write(0, 0)
  issue_inputs(2, 0)
  wait_idx(0)
  issue_gathers(0)
  wait_gathers_c(1)
  compute_write(1, 1)
  issue_inputs(3, 1)

  def steady(m, carry):
    k0 = 2 * m + 2
    wait_idx(1)
    issue_gathers(1)
    wait_write(0)
    wait_gathers_c(0)
    compute_write(k0, 0)
    issue_inputs(k0 + 2, 0)
    wait_idx(0)
    issue_gathers(0)
    wait_write(1)
    wait_gathers_c(1)
    compute_write(k0 + 1, 1)
    issue_inputs(k0 + 3, 1)
    return carry

  lax.fori_loop(0, (ROUNDS_W - 4) // 2, steady, 0)

  wait_idx(1)
  issue_gathers(1)
  wait_write(0)
  wait_gathers_c(0)
  compute_write(ROUNDS_W - 2, 0)
  wait_write(1)
  wait_gathers_c(1)
  compute_write(ROUNDS_W - 1, 1)
  wait_write(0)
  wait_write(1)

  pltpu.sync_copy(sqb, sq_hbm.at[wid])


def _sc_combine(ab_tbl, cp, dst, src):
  mesh = plsc.VectorSubcoreMesh(core_axis_name="c", subcore_axis_name="s",
                                num_cores=NC, num_subcores=NS)
  f = pl.kernel(
      _sc_combine_body,
      out_type=[
          jax.ShapeDtypeStruct((PCH, 128), jnp.float32),
          jax.ShapeDtypeStruct((NW, 8, 128), jnp.float32),
      ],
      mesh=mesh,
      scratch_types=(
          [pltpu.VMEM((CHUNK,), jnp.int32)] * 4
          + [pltpu.VMEM((CHUNK, 128), jnp.float32)] * 4
          + [pltpu.VMEM((CHUNK // 2, 128), jnp.float32)] * 4
          + [pltpu.VMEM((8, 128), jnp.float32)]
          + [pltpu.SemaphoreType.DMA] * 12
      ),
  )
  return f(ab_tbl, cp, dst, src)


def _sc_scatter_body(y_hbm, dst_hbm, ac_hbm, out_hbm,
                     tbl, idxe0, idxe1, idxo0, idxo1, ybuf0, ybuf1,
                     rowsl, rowsr, zb, acb,
                     sld0, sld1, sy0, sy1, ssc):
  c = lax.axis_index("c")
  s = lax.axis_index("s")
  lo = c * HALF
  idxe = [idxe0, idxe1]
  idxo = [idxo0, idxo1]
  ybuf = [ybuf0, ybuf1]
  sld = [sld0, sld1]
  sy = [sy0, sy1]

  def zrow(j, carry):
    for g in range(8):
      sl = pl.ds(g * 16, 16)
      zb[j, sl] = jnp.zeros((16,), jnp.float32)
    return carry
  lax.fori_loop(0, ZCH, zrow, 0)

  def zrow2(j, carry):
    for g in range(8):
      sl = pl.ds(g * 16, 16)
      rowsl[j, sl] = jnp.zeros((16,), jnp.float32)
      rowsr[j, sl] = jnp.zeros((16,), jnp.float32)
    return carry
  lax.fori_loop(0, SCHUNK, zrow2, 0)

  def zcopy(k, carry):
    pltpu.sync_copy(zb, tbl.at[pl.ds((s * ZCOPIES + k) * ZCH, ZCH)])
    return carry
  lax.fori_loop(0, ZCOPIES, zcopy, 0)

  pltpu.sync_copy(ac_hbm, acb)

  plsc.subcore_barrier()

  a_regs = [acb[0, pl.ds(g * 16, 16)] for g in range(4)]
  c_regs = [acb[1, pl.ds(g * 16, 16)] for g in range(4)]
  lane = lax.iota(jnp.int32, 16)

  def issue_loads(i, p):
    cid = i * NS + s
    base = cid * SCHUNK
    pbase = cid * (SCHUNK // 2)
    pltpu.async_copy(dst_hbm.at[pl.ds(base, SCHUNK)], idxe[p], sld[p])
    pltpu.async_copy(y_hbm.at[pl.ds(pbase, SCHUNK // 2)], ybuf[p], sy[p])

  def wait_loads(p):
    pltpu.make_async_copy(dst_hbm.at[pl.ds(0, SCHUNK)], idxe[p],
                          sld[p]).wait()
    pltpu.make_async_copy(y_hbm.at[pl.ds(0, SCHUNK // 2)], ybuf[p],
                          sy[p]).wait()

  def wait_scatters(p):
    pltpu.make_async_copy(rowsl, tbl.at[idxe[p]], ssc).wait()
    pltpu.make_async_copy(rowsr, tbl.at[idxo[p]], ssc).wait()

  def compute_scatter(p):
    for jj in range(SCHUNK // 16):
      sl = pl.ds(jj * 16, 16)
      v = idxe[p][sl]
      valid = (v >= lo) & (v < lo + HALF)
      local = v - lo
      packed = lax.shift_right_logical(local, 1)
      oddb = lax.bitwise_and(local, 1)
      dump = TBL_P + lane + 16 * jj
      dump2 = dump + 64
      evens = packed + oddb * (dump - packed)
      odds = packed + (1 - oddb) * (dump2 - packed)
      idxe[p][sl] = jnp.where(valid, evens, dump)
      idxo[p][sl] = jnp.where(valid, odds, dump2)

    def row_body(r, carry2):
      for par in range(2):
        j = 2 * r + par
        for g in range(4):
          v = ybuf[p][r, pl.ds(par * D + g * 16, 16)]
          mv = jnp.maximum(v * a_regs[g] + c_regs[g], 0.0)
          rowsl[j, pl.ds(g * 16, 16)] = mv
          rowsr[j, pl.ds(D + g * 16, 16)] = mv
      return carry2
    lax.fori_loop(0, SCHUNK // 2, row_body, 0)

    pltpu.async_copy(rowsl, tbl.at[idxe[p]], ssc, add=True)
    pltpu.async_copy(rowsr, tbl.at[idxo[p]], ssc, add=True)

  issue_loads(0, 0)
  wait_loads(0)
  issue_loads(1, 1)
  compute_scatter(0)

  def steady(m, carry):
    i0 = 2 * m + 1
    wait_loads(1)
    wait_scatters(0)
    issue_loads(i0 + 1, 0)
    compute_scatter(1)
    wait_loads(0)
    wait_scatters(1)
    issue_loads(i0 + 2, 1)
    compute_scatter(0)
    return carry

  lax.fori_loop(0, (SROUNDS - 2) // 2, steady, 0)

  wait_loads(1)
  wait_scatters(0)
  compute_scatter(1)
  wait_scatters(1)

  plsc.subcore_barrier()

  def drain_body(k, carry):
    prb = s * (TBL_PV // NS) + k * DR_P
    pltpu.sync_copy(tbl.at[pl.ds(prb, DR_P)],
                    out_hbm.at[pl.ds(c * TBL_PV + prb, DR_P)])
    return carry

  lax.fori_loop(0, DR_PER_TILE, drain_body, 0)


def _sc_scatter(y_edge, dst, ac):
  mesh = plsc.VectorSubcoreMesh(core_axis_name="c", subcore_axis_name="s",
                                num_cores=NC, num_subcores=NS)
  f = pl.kernel(
      _sc_scatter_body,
      out_type=jax.ShapeDtypeStruct((2 * TBL_PV, 128), jnp.float32),
      mesh=mesh,
      scratch_types=(
          [pltpu.VMEM_SHARED((STBL, 128), jnp.float32)]
          + [pltpu.VMEM((SCHUNK,), jnp.int32)] * 4
          + [pltpu.VMEM((SCHUNK // 2, 128), jnp.float32)] * 2
          + [pltpu.VMEM((SCHUNK, 128), jnp.float32)] * 2
          + [pltpu.VMEM((ZCH, 128), jnp.float32)]
          + [pltpu.VMEM((8, 128), jnp.float32)]
          + [pltpu.SemaphoreType.DMA] * 5
      ),
  )
  return f(y_edge, dst, ac)


def _pad_gb(g, b):
  gb = jnp.zeros((8, D), jnp.float32)
  return gb.at[0, :].set(g).at[1, :].set(b)


def kernel(x, edge_attr, prev_h, params, edge_index):
  pad = jnp.full((EPAD - E,), N, jnp.int32)
  src = jnp.concatenate([edge_index[0], pad])
  dst = jnp.concatenate([edge_index[1], pad])

  xp = x.reshape(NP, 2 * IN_DIM)
  hprevp = prev_h.reshape(NP, 128)
  eap = jnp.zeros((PCH, 2 * ED), jnp.float32).at[:EP].set(
      edge_attr.reshape(EP, 2 * ED))

  b_in_hist = params['b_in'] + params['b_hist']
  b128 = jnp.tile(b_in_hist, 2)[None, :]

  hp = _input_proj(xp, hprevp, _bd2(params['W_in']), _bd2(params['W_hist']),
                   b128)

  for l in range(L):
    w1 = params['msg_W1'][l]
    w1a, w1b, w1c = w1[:D], w1[D:2 * D], w1[2 * D:]

    wab = jnp.concatenate([w1a, w1b], axis=1)
    w_e = jnp.zeros((128, 128), jnp.float32).at[:D, :].set(wab)
    w_o = jnp.zeros((128, 128), jnp.float32).at[D:, :].set(wab)

    ab_tbl = _precompute_tbl(hp, w_e, w_o)
    tblp = jnp.zeros((N + 8, 128), jnp.float32).at[:N].set(ab_tbl)
    cp = _edge_c(eap, _bd2(w1c))

    y1, sq1 = _sc_combine(tblp, cp, dst, src)

    ac1 = _finalize(sq1,
                    _pad_gb(params['msg_g1'][l], params['msg_be1'][l]), E)
    y2p, sq2 = _bn_mm_stats(y1, ac1,
                            _bd2(params['msg_W2'][l]), PCH, RB_E,
                            valid_rows=EP)
    ac2 = _finalize_sq(sq2,
                       _pad_gb(params['msg_g2'][l], params['msg_be2'][l]), E)

    slab = _sc_scatter(y2p, dst, ac2)
    aggrp = jnp.concatenate(
        [slab[:HALF // 2], slab[TBL_PV:TBL_PV + HALF // 2]], axis=0)

    u1 = params['upd_W1'][l]
    y3p, sq3 = _update1(hp, aggrp, _bd2(u1[:D]), _bd2(u1[D:]))
    ac3 = _finalize_sq(sq3,
                       _pad_gb(params['upd_g1'][l], params['upd_be1'][l]), N)
    y4p, sq4 = _bn_mm_stats(y3p, ac3, _bd2(params['upd_W2'][l]), NP, RB_N)
    ac4 = _finalize_sq(sq4,
                       _pad_gb(params['upd_g2'][l], params['upd_be2'][l]), N)
    hp = _resid(hp, y4p, ac4)

  wn_bd = _bd2(params['Wn'])
  nb128 = jnp.tile(params['bn'], 2)[None, :]
  node_out, hs = _heads(hp, wn_bd, nb128)

  gp = jnp.zeros((8, D), jnp.float32)
  gp = gp.at[0, :].set(params['Wg'][:, 0]).at[1, 0].set(params['bg'][0])
  graph_out = _graph_head(hs, gp)

  return (graph_out, node_out, hp.reshape(N, D))
